# baseline (speedup 1.0000x reference)
import functools

import jax
import jax.numpy as jnp
from jax import lax
from jax.experimental import pallas as pl
from jax.experimental.pallas import tpu as pltpu
from jax.experimental.pallas import tpu_sc as plsc

N = 10000
E = 320000
D = 128
DE = 16
H = 128

NP = 10240
NCORE = 2
NSUB = 16
NW = NCORE * NSUB
EPW = E // NW
CCH = 40
NCCH = EPW // CCH
CH = 80
NCHUNK = EPW // CH
NSLOT = 5
RPT = NP // NSUB

BN = 512
BEDGE = 2000

_MESH = plsc.VectorSubcoreMesh(core_axis_name="c", subcore_axis_name="s")
_SC_PARAMS = pltpu.CompilerParams(use_tc_tiling_on_sc=False)


def _conv_sc_body(ts_hbm, td_hbm, pe_hbm, src_hbm, dst_hbm, wg2_hbm, bg2_hbm,
                  out_hbm,
                  idx_s0, idx_d0, idx_s1, idx_d1, sidx0, sidx1,
                  gs0, gd0, gs1, gd1, pe, msg0, msg1, wg2v, bg2v, acc,
                  semg0, semg1, semc0, semc1):
    c = lax.axis_index("c")
    s = lax.axis_index("s")
    wid = c * NSUB + s

    idx_s = [idx_s0, idx_s1]
    idx_d = [idx_d0, idx_d1]
    sidx = [sidx0, sidx1]
    gs = [gs0, gs1]
    gd = [gd0, gd1]
    msg = [msg0, msg1]
    semg = [semg0, semg1]
    semc = [semc0, semc1]

    zero16 = jnp.zeros((16,), jnp.float32)

    def zrow(i, carry):
        for k in range(8):
            msg0[i, pl.ds(16 * k, 16)] = zero16
        return carry

    lax.fori_loop(0, CCH, zrow, 0)
    rbase = s * RPT
    for t in range(RPT // CCH):
        pltpu.sync_copy(msg0, acc.at[pl.ds(rbase + t * CCH, CCH)])
    plsc.subcore_barrier()

    pltpu.sync_copy(wg2_hbm, wg2v)
    pltpu.sync_copy(bg2_hbm, bg2v)
    wg2k = [wg2v[pl.ds(16 * k, 16)] for k in range(4)]
    bg2 = bg2v[...]
    lane = lax.iota(jnp.int32, 16)
    gdn = lax.GatherDimensionNumbers(
        offset_dims=(), collapsed_slice_dims=(0,), start_index_map=(0,))
    perms = [jnp.bitwise_and(lane + sh, 15)[:, None] for sh in (8, 4, 2, 1)]

    ebase = wid * EPW

    def issue(ci, b):
        cb = ebase + ci * CCH
        pltpu.sync_copy(src_hbm.at[pl.ds(cb, CCH)], idx_s[b])
        pltpu.sync_copy(dst_hbm.at[pl.ds(cb, CCH)], idx_d[b])
        pltpu.async_copy(ts_hbm.at[idx_s[b]], gs[b], semg[b])
        pltpu.async_copy(td_hbm.at[idx_d[b]], gd[b], semg[b])

    def wait_gathers(b):
        pltpu.make_async_copy(ts_hbm.at[idx_s[b]], gs[b], semg[b]).wait()
        pltpu.make_async_copy(td_hbm.at[idx_d[b]], gd[b], semg[b]).wait()

    def wait_scatter(b):
        pltpu.make_async_copy(msg[b], acc.at[sidx[b]], semc[b]).wait()

    def copy_idx(b):
        sidx[b][pl.ds(0, 16)] = idx_d[b][pl.ds(0, 16)]
        sidx[b][pl.ds(16, 16)] = idx_d[b][pl.ds(16, 16)]
        sidx[b][pl.ds(24, 16)] = idx_d[b][pl.ds(24, 16)]

    def edge_one(e2, b):
        gh_acc = None
        for k in range(4):
            ghk = (gs[b][e2, pl.ds(128 + 16 * k, 16)]
                   + gd[b][e2, pl.ds(128 + 16 * k, 16)]
                   + pe[e2, pl.ds(128 + 16 * k, 16)])
            ghk = jnp.maximum(ghk, 0.0) * wg2k[k]
            gh_acc = ghk if gh_acc is None else gh_acc + ghk
        v = gh_acc
        for pm in perms:
            v = v + lax.gather(
                v, pm, gdn, slice_sizes=(1,),
                mode=lax.GatherScatterMode.PROMISE_IN_BOUNDS)
        tv = v + bg2
        gate = 1.0 / (1.0 + jnp.exp(-tv))
        for k in range(8):
            raw = (gs[b][e2, pl.ds(16 * k, 16)]
                   + gd[b][e2, pl.ds(16 * k, 16)]
                   + pe[e2, pl.ds(16 * k, 16)])
            msg[b][e2, pl.ds(16 * k, 16)] = jnp.maximum(raw, 0.0) * gate

    def halfstep(i, ci, b):
        wait_gathers(b)
        pltpu.sync_copy(pe_hbm.at[pl.ds(ebase + ci * CCH, CCH)], pe)

        @pl.when(i > 0)
        def _():
            wait_scatter(b)

        @plsc.parallel_loop(0, CCH, 1, unroll=8)
        def _(e2):
            edge_one(e2, b)

        copy_idx(b)
        pltpu.async_copy(msg[b], acc.at[sidx[b]], semc[b], add=True)

        @pl.when(i < NCCH // 2 - 1)
        def _():
            issue(ci + 2, b)

    issue(0, 0)
    issue(1, 1)

    def chunk_pair(i, carry):
        halfstep(i, 2 * i, 0)
        halfstep(i, 2 * i + 1, 1)
        return carry

    lax.fori_loop(0, NCCH // 2, chunk_pair, 0)
    wait_scatter(0)
    wait_scatter(1)
    plsc.subcore_barrier()
    pltpu.sync_copy(acc.at[pl.ds(rbase, RPT)], out_hbm.at[c, pl.ds(rbase, RPT)])


_conv_sc = pl.kernel(
    _conv_sc_body,
    out_type=jax.ShapeDtypeStruct((NCORE, NP, 128), jnp.float32),
    mesh=_MESH,
    scratch_types=[
        pltpu.VMEM((CCH,), jnp.int32),
        pltpu.VMEM((CCH,), jnp.int32),
        pltpu.VMEM((CCH,), jnp.int32),
        pltpu.VMEM((CCH,), jnp.int32),
        pltpu.VMEM((CCH,), jnp.int32),
        pltpu.VMEM((CCH,), jnp.int32),
        pltpu.VMEM((CCH, 192), jnp.float32),
        pltpu.VMEM((CCH, 192), jnp.float32),
        pltpu.VMEM((CCH, 192), jnp.float32),
        pltpu.VMEM((CCH, 192), jnp.float32),
        pltpu.VMEM((CCH, 192), jnp.float32),
        pltpu.VMEM((CCH, 128), jnp.float32),
        pltpu.VMEM((CCH, 128), jnp.float32),
        pltpu.VMEM((64,), jnp.float32),
        pltpu.VMEM((16,), jnp.float32),
        pltpu.VMEM_SHARED((NP, 128), jnp.float32),
        pltpu.SemaphoreType.DMA,
        pltpu.SemaphoreType.DMA,
        pltpu.SemaphoreType.DMA,
        pltpu.SemaphoreType.DMA,
    ],
    compiler_params=_SC_PARAMS,
)


def _count_sc_body(dst_hbm, out_hbm, idx_d, ones, acc, sem0):
    c = lax.axis_index("c")
    s = lax.axis_index("s")
    wid = c * NSUB + s

    zero16 = jnp.zeros((16,), jnp.float32)

    def zrow(i, carry):
        ones[i, pl.ds(0, 16)] = zero16
        return carry

    lax.fori_loop(0, CH, zrow, 0)
    rbase = s * RPT
    for t in range(RPT // CH):
        pltpu.sync_copy(ones, acc.at[pl.ds(rbase + t * CH, CH)])
    plsc.subcore_barrier()

    lane = lax.iota(jnp.int32, 16)
    cnt_vec = jnp.where(lane == 0, 1.0, 0.0).astype(jnp.float32)

    def orow(i, carry):
        ones[i, pl.ds(0, 16)] = cnt_vec
        return carry

    lax.fori_loop(0, CH, orow, 0)

    ebase = wid * EPW

    def chunk_body(i, carry):
        cb = ebase + i * CH
        pltpu.sync_copy(dst_hbm.at[pl.ds(cb, CH)], idx_d)
        pltpu.sync_copy(ones, acc.at[idx_d], add=True)
        return carry

    lax.fori_loop(0, NCHUNK, chunk_body, 0)
    plsc.subcore_barrier()
    pltpu.sync_copy(acc.at[pl.ds(rbase, RPT)], out_hbm.at[c, pl.ds(rbase, RPT)])


_count_sc = pl.kernel(
    _count_sc_body,
    out_type=jax.ShapeDtypeStruct((NCORE, NP, 16), jnp.float32),
    mesh=_MESH,
    scratch_types=[
        pltpu.VMEM((CH,), jnp.int32),
        pltpu.VMEM((CH, 16), jnp.float32),
        pltpu.VMEM_SHARED((NP, 16), jnp.float32),
        pltpu.SemaphoreType.DMA,
    ],
    compiler_params=_SC_PARAMS,
)


def _clf_sc_body(ht_hbm, src_hbm, dst_hbm, zs_hbm, zd_hbm,
                 idx_s0, idx_d0, idx_s1, idx_d1, idx_s2, idx_d2,
                 idx_s3, idx_d3, idx_s4, idx_d4,
                 u0, v0, u1, v1, u2, v2, u3, v3, u4, v4,
                 semg0, semg1, semg2, semg3, semg4,
                 semw0, semw1, semw2, semw3, semw4):
    c = lax.axis_index("c")
    s = lax.axis_index("s")
    wid = c * NSUB + s
    ebase = wid * EPW

    idx_s = [idx_s0, idx_s1, idx_s2, idx_s3, idx_s4]
    idx_d = [idx_d0, idx_d1, idx_d2, idx_d3, idx_d4]
    u = [u0, u1, u2, u3, u4]
    v = [v0, v1, v2, v3, v4]
    semg = [semg0, semg1, semg2, semg3, semg4]
    semw = [semw0, semw1, semw2, semw3, semw4]

    def issue(ci, b):
        cb = ebase + ci * CH
        pltpu.sync_copy(src_hbm.at[pl.ds(cb, CH)], idx_s[b])
        pltpu.sync_copy(dst_hbm.at[pl.ds(cb, CH)], idx_d[b])
        pltpu.async_copy(ht_hbm.at[idx_s[b]], u[b], semg[b])
        pltpu.async_copy(ht_hbm.at[idx_d[b]], v[b], semg[b])

    def wait_gathers(b):
        pltpu.make_async_copy(ht_hbm.at[idx_s[b]], u[b], semg[b]).wait()
        pltpu.make_async_copy(ht_hbm.at[idx_d[b]], v[b], semg[b]).wait()

    def start_writes(ci, b):
        cb = ebase + ci * CH
        pltpu.async_copy(u[b], zs_hbm.at[pl.ds(cb, CH)], semw[b])
        pltpu.async_copy(v[b], zd_hbm.at[pl.ds(cb, CH)], semw[b])

    def wait_writes(ci, b):
        cb = ebase + ci * CH
        pltpu.make_async_copy(u[b], zs_hbm.at[pl.ds(cb, CH)], semw[b]).wait()
        pltpu.make_async_copy(v[b], zd_hbm.at[pl.ds(cb, CH)], semw[b]).wait()

    for b in range(NSLOT):
        issue(b, b)

    nq = NCHUNK // NSLOT

    def quint(i, carry):
        for b in range(NSLOT):
            wait_gathers(b)
            start_writes(i * NSLOT + b, b)
        for b in range(NSLOT):
            @pl.when(i < nq - 1)
            def _(b=b):
                wait_writes(i * NSLOT + b, b)
                issue((i + 1) * NSLOT + b, b)
        return carry

    lax.fori_loop(0, nq, quint, 0)
    for b in range(NSLOT):
        wait_writes((nq - 1) * NSLOT + b, b)


_clf_sc = pl.kernel(
    _clf_sc_body,
    out_type=(jax.ShapeDtypeStruct((E, 128), jnp.float32),
              jax.ShapeDtypeStruct((E, 128), jnp.float32)),
    mesh=_MESH,
    scratch_types=(
        [pltpu.VMEM((CH,), jnp.int32) for _ in range(2 * NSLOT)]
        + [pltpu.VMEM((CH, 128), jnp.float32) for _ in range(2 * NSLOT)]
        + [pltpu.SemaphoreType.DMA for _ in range(2 * NSLOT)]
    ),
    compiler_params=_SC_PARAMS,
)


def _node_pre_body(x_ref, w1_ref, w2_ref, w3_ref, b3_ref,
                   ts_ref, td_ref, self_ref):
    x = x_ref[...]
    ts_ref[...] = jnp.dot(x, w1_ref[...], preferred_element_type=jnp.float32)
    td_ref[...] = jnp.dot(x, w2_ref[...], preferred_element_type=jnp.float32)
    self_ref[...] = (
        jnp.dot(x, w3_ref[...], preferred_element_type=jnp.float32)
        + b3_ref[...])


def _node_pre(x, w1, w2, w3, b3):
    return pl.pallas_call(
        _node_pre_body,
        grid=(NP // BN,),
        in_specs=[
            pl.BlockSpec((BN, 128), lambda i: (i, 0)),
            pl.BlockSpec((128, 192), lambda i: (0, 0)),
            pl.BlockSpec((128, 192), lambda i: (0, 0)),
            pl.BlockSpec((128, 128), lambda i: (0, 0)),
            pl.BlockSpec((1, 128), lambda i: (0, 0)),
        ],
        out_specs=[
            pl.BlockSpec((BN, 192), lambda i: (i, 0)),
            pl.BlockSpec((BN, 192), lambda i: (i, 0)),
            pl.BlockSpec((BN, 128), lambda i: (i, 0)),
        ],
        out_shape=[
            jax.ShapeDtypeStruct((NP, 192), jnp.float32),
            jax.ShapeDtypeStruct((NP, 192), jnp.float32),
            jax.ShapeDtypeStruct((NP, 128), jnp.float32),
        ],
    )(x, w1, w2, w3, b3)


def _edge_pre_body(e_ref, w0_ref, b0_ref, w1_ref, b1_ref,
                   p0_ref, p1_ref):
    e = e_ref[...]
    p0_ref[...] = (
        jnp.dot(e, w0_ref[...], preferred_element_type=jnp.float32)
        + b0_ref[...])
    p1_ref[...] = (
        jnp.dot(e, w1_ref[...], preferred_element_type=jnp.float32)
        + b1_ref[...])


def _edge_pre(e, w0, b0, w1, b1):
    return pl.pallas_call(
        _edge_pre_body,
        grid=(E // BEDGE,),
        in_specs=[
            pl.BlockSpec((BEDGE, 16), lambda i: (i, 0)),
            pl.BlockSpec((16, 192), lambda i: (0, 0)),
            pl.BlockSpec((1, 192), lambda i: (0, 0)),
            pl.BlockSpec((16, 192), lambda i: (0, 0)),
            pl.BlockSpec((1, 192), lambda i: (0, 0)),
        ],
        out_specs=[
            pl.BlockSpec((BEDGE, 192), lambda i: (i, 0)),
            pl.BlockSpec((BEDGE, 192), lambda i: (i, 0)),
        ],
        out_shape=[
            jax.ShapeDtypeStruct((E, 192), jnp.float32),
            jax.ShapeDtypeStruct((E, 192), jnp.float32),
        ],
    )(e, w0, b0, w1, b1)


def _finish_node(a0, a1, c0, c1, slf, g, b):
    sums = a0 + a1
    cnt = jnp.sum(c0 + c1, axis=1, keepdims=True)
    agg = sums / jnp.maximum(cnt, 1.0)
    o = agg + slf
    m = jnp.mean(o, axis=-1, keepdims=True)
    v = jnp.mean((o - m) ** 2, axis=-1, keepdims=True)
    hn = (o - m) / jnp.sqrt(v + 1e-5) * g + b
    return jnp.maximum(hn, 0.0)


def _combine0_body(a0_ref, a1_ref, c0_ref, c1_ref, self_ref, g_ref, b_ref,
                   w1_ref, w2_ref, w3_ref, b3_ref,
                   ts_ref, td_ref, self1_ref):
    h = _finish_node(a0_ref[...], a1_ref[...], c0_ref[...], c1_ref[...],
                     self_ref[...], g_ref[...], b_ref[...])
    ts_ref[...] = jnp.dot(h, w1_ref[...], preferred_element_type=jnp.float32)
    td_ref[...] = jnp.dot(h, w2_ref[...], preferred_element_type=jnp.float32)
    self1_ref[...] = (
        jnp.dot(h, w3_ref[...], preferred_element_type=jnp.float32)
        + b3_ref[...])


def _combine0(a0, a1, c0, c1, slf, g, b, w1, w2, w3, b3):
    return pl.pallas_call(
        _combine0_body,
        grid=(NP // BN,),
        in_specs=[
            pl.BlockSpec((BN, 128), lambda i: (i, 0)),
            pl.BlockSpec((BN, 128), lambda i: (i, 0)),
            pl.BlockSpec((BN, 16), lambda i: (i, 0)),
            pl.BlockSpec((BN, 16), lambda i: (i, 0)),
            pl.BlockSpec((BN, 128), lambda i: (i, 0)),
            pl.BlockSpec((1, 128), lambda i: (0, 0)),
            pl.BlockSpec((1, 128), lambda i: (0, 0)),
            pl.BlockSpec((128, 192), lambda i: (0, 0)),
            pl.BlockSpec((128, 192), lambda i: (0, 0)),
            pl.BlockSpec((128, 128), lambda i: (0, 0)),
            pl.BlockSpec((1, 128), lambda i: (0, 0)),
        ],
        out_specs=[
            pl.BlockSpec((BN, 192), lambda i: (i, 0)),
            pl.BlockSpec((BN, 192), lambda i: (i, 0)),
            pl.BlockSpec((BN, 128), lambda i: (i, 0)),
        ],
        out_shape=[
            jax.ShapeDtypeStruct((NP, 192), jnp.float32),
            jax.ShapeDtypeStruct((NP, 192), jnp.float32),
            jax.ShapeDtypeStruct((NP, 128), jnp.float32),
        ],
    )(a0, a1, c0, c1, slf, g, b, w1, w2, w3, b3)


def _combine1_body(a0_ref, a1_ref, c0_ref, c1_ref, self_ref, g_ref, b_ref,
                   hp_ref, wih_ref, bih_ref, whh_ref, bhh_ref,
                   ht_ref):
    h = _finish_node(a0_ref[...], a1_ref[...], c0_ref[...], c1_ref[...],
                     self_ref[...], g_ref[...], b_ref[...])
    hp = hp_ref[...]
    gi = jnp.dot(h, wih_ref[...], preferred_element_type=jnp.float32) \
        + bih_ref[...]
    gh = jnp.dot(hp, whh_ref[...], preferred_element_type=jnp.float32) \
        + bhh_ref[...]
    r = jax.nn.sigmoid(gi[:, :128] + gh[:, :128])
    z = jax.nn.sigmoid(gi[:, 128:256] + gh[:, 128:256])
    n = jnp.tanh(gi[:, 256:384] + r * gh[:, 256:384])
    ht_ref[...] = (1.0 - z) * n + z * hp


def _combine1(a0, a1, c0, c1, slf, g, b, hp, wih, bih, whh, bhh):
    return pl.pallas_call(
        _combine1_body,
        grid=(NP // BN,),
        in_specs=[
            pl.BlockSpec((BN, 128), lambda i: (i, 0)),
            pl.BlockSpec((BN, 128), lambda i: (i, 0)),
            pl.BlockSpec((BN, 16), lambda i: (i, 0)),
            pl.BlockSpec((BN, 16), lambda i: (i, 0)),
            pl.BlockSpec((BN, 128), lambda i: (i, 0)),
            pl.BlockSpec((1, 128), lambda i: (0, 0)),
            pl.BlockSpec((1, 128), lambda i: (0, 0)),
            pl.BlockSpec((BN, 128), lambda i: (i, 0)),
            pl.BlockSpec((128, 384), lambda i: (0, 0)),
            pl.BlockSpec((1, 384), lambda i: (0, 0)),
            pl.BlockSpec((128, 384), lambda i: (0, 0)),
            pl.BlockSpec((1, 384), lambda i: (0, 0)),
        ],
        out_specs=pl.BlockSpec((BN, 128), lambda i: (i, 0)),
        out_shape=jax.ShapeDtypeStruct((NP, 128), jnp.float32),
    )(a0, a1, c0, c1, slf, g, b, hp, wih, bih, whh, bhh)


def _head_body(zs_ref, zd_ref, e_ref, wem_ref, bm1_ref,
               ws_ref, wd_ref, wa_ref, wp_ref,
               wm2_ref, bm2_ref, out_ref):
    zs = zs_ref[...]
    zd = zd_ref[...]
    hm = (jnp.dot(e_ref[...], wem_ref[...],
                  preferred_element_type=jnp.float32)
          + bm1_ref[...])
    hm = hm + jnp.dot(zs, ws_ref[...], preferred_element_type=jnp.float32)
    hm = hm + jnp.dot(zd, wd_ref[...], preferred_element_type=jnp.float32)
    hm = hm + jnp.dot(jnp.abs(zs - zd), wa_ref[...],
                      preferred_element_type=jnp.float32)
    hm = hm + jnp.dot(zs * zd, wp_ref[...],
                      preferred_element_type=jnp.float32)
    hm = jnp.maximum(hm, 0.0)
    out_ref[...] = (
        jnp.dot(hm, wm2_ref[...], preferred_element_type=jnp.float32)
        + bm2_ref[...])


def _head(zs, zd, e, wem_t, bm1, ws_t, wd_t, wa_t, wp_t, wm2_t, bm2):
    return pl.pallas_call(
        _head_body,
        grid=(E // BEDGE,),
        in_specs=[
            pl.BlockSpec((BEDGE, 128), lambda i: (i, 0)),
            pl.BlockSpec((BEDGE, 128), lambda i: (i, 0)),
            pl.BlockSpec((BEDGE, 16), lambda i: (i, 0)),
            pl.BlockSpec((16, 128), lambda i: (0, 0)),
            pl.BlockSpec((1, 128), lambda i: (0, 0)),
            pl.BlockSpec((128, 128), lambda i: (0, 0)),
            pl.BlockSpec((128, 128), lambda i: (0, 0)),
            pl.BlockSpec((128, 128), lambda i: (0, 0)),
            pl.BlockSpec((128, 128), lambda i: (0, 0)),
            pl.BlockSpec((128, 2), lambda i: (0, 0)),
            pl.BlockSpec((1, 2), lambda i: (0, 0)),
        ],
        out_specs=pl.BlockSpec((BEDGE, 2), lambda i: (i, 0)),
        out_shape=jax.ShapeDtypeStruct((E, 2), jnp.float32),
    )(zs, zd, e, wem_t, bm1, ws_t, wd_t, wa_t, wp_t, wm2_t, bm2)


def kernel(x, edge_index, edge_attr, h_prev, params):
    p = params
    src = edge_index[0]
    dst = edge_index[1]
    e = edge_attr

    x_pad = jnp.zeros((NP, D), jnp.float32).at[:N].set(x)
    hp_pad = jnp.zeros((NP, H), jnp.float32).at[:N].set(h_prev)

    wg1_0, wg1_1 = p['Wg1_0'], p['Wg1_1']
    w_p0 = jnp.concatenate([p['W_edge0'].T, wg1_0[:, 2 * D:].T], axis=1)
    b_p0 = jnp.concatenate([jnp.zeros((128,), jnp.float32), p['bg1_0']])
    w_p1 = jnp.concatenate([p['W_edge1'].T, wg1_1[:, 2 * D:].T], axis=1)
    b_p1 = jnp.concatenate([jnp.zeros((128,), jnp.float32), p['bg1_1']])
    wm1 = p['Wm1']
    pe0, pe1 = _edge_pre(e, w_p0, b_p0.reshape(1, -1),
                         w_p1, b_p1.reshape(1, -1))

    cnt = _count_sc(dst)

    w1 = jnp.concatenate([p['W_src0'].T, wg1_0[:, D:2 * D].T], axis=1)
    w2 = jnp.concatenate([p['W_dst0'].T, wg1_0[:, :D].T], axis=1)
    ts0, td0, slf0 = _node_pre(x_pad, w1, w2, p['W_self0'].T,
                               p['b_self0'].reshape(1, -1))

    wg2_0 = p['Wg2_0'].reshape(64)
    bg2_0 = jnp.broadcast_to(p['bg2_0'].reshape(1), (16,))
    acc0 = _conv_sc(ts0, td0, pe0, src, dst, wg2_0, bg2_0)

    w1b = jnp.concatenate([p['W_src1'].T, wg1_1[:, D:2 * D].T], axis=1)
    w2b = jnp.concatenate([p['W_dst1'].T, wg1_1[:, :D].T], axis=1)
    ts1, td1, slf1 = _combine0(acc0[0], acc0[1], cnt[0], cnt[1], slf0,
                               p['ln_g0'].reshape(1, -1),
                               p['ln_b0'].reshape(1, -1),
                               w1b, w2b, p['W_self1'].T,
                               p['b_self1'].reshape(1, -1))

    wg2_1 = p['Wg2_1'].reshape(64)
    bg2_1 = jnp.broadcast_to(p['bg2_1'].reshape(1), (16,))
    acc1 = _conv_sc(ts1, td1, pe1, src, dst, wg2_1, bg2_1)

    ht_pad = _combine1(acc1[0], acc1[1], cnt[0], cnt[1], slf1,
                       p['ln_g1'].reshape(1, -1),
                       p['ln_b1'].reshape(1, -1),
                       hp_pad,
                       p['W_ih'].T, p['b_ih'].reshape(1, -1),
                       p['W_hh'].T, p['b_hh'].reshape(1, -1))

    zs, zd = _clf_sc(ht_pad, src, dst)

    logits = _head(zs, zd, e,
                   wm1[:, 2 * H:2 * H + DE].T, p['bm1'].reshape(1, -1),
                   wm1[:, :H].T, wm1[:, H:2 * H].T,
                   wm1[:, 2 * H + DE:3 * H + DE].T, wm1[:, 3 * H + DE:].T,
                   p['Wm2'].T, p['bm2'].reshape(1, 2))
    return logits, ht_pad[:N]


# speedup vs baseline: 1.0175x; 1.0175x over previous
import functools

import jax
import jax.numpy as jnp
from jax import lax
from jax.experimental import pallas as pl
from jax.experimental.pallas import tpu as pltpu
from jax.experimental.pallas import tpu_sc as plsc

N = 10000
E = 320000
D = 128
DE = 16
H = 128

NP = 10240
NCORE = 2
NSUB = 16
NW = NCORE * NSUB
EPW = E // NW
CCH = 40
NCCH = EPW // CCH
CH = 80
NCHUNK = EPW // CH
NSLOT = 5
RPT = NP // NSUB

BN = 512
BEDGE = 2000

_MESH = plsc.VectorSubcoreMesh(core_axis_name="c", subcore_axis_name="s")
_SC_PARAMS = pltpu.CompilerParams(use_tc_tiling_on_sc=False)


def _conv_sc_body(ts_hbm, td_hbm, pe_hbm, src_hbm, dst_hbm, wg2_hbm, bg2_hbm,
                  out_hbm,
                  idx_s0, idx_d0, idx_s1, idx_d1, sidx0, sidx1,
                  gs0, gd0, gs1, gd1, pe, msg0, msg1, wg2v, bg2v, acc,
                  semg0, semg1, semc0, semc1):
    c = lax.axis_index("c")
    s = lax.axis_index("s")
    wid = c * NSUB + s

    idx_s = [idx_s0, idx_s1]
    idx_d = [idx_d0, idx_d1]
    sidx = [sidx0, sidx1]
    gs = [gs0, gs1]
    gd = [gd0, gd1]
    msg = [msg0, msg1]
    semg = [semg0, semg1]
    semc = [semc0, semc1]

    zero16 = jnp.zeros((16,), jnp.float32)

    def zrow(i, carry):
        for k in range(8):
            msg0[i, pl.ds(16 * k, 16)] = zero16
        return carry

    lax.fori_loop(0, CCH, zrow, 0)
    rbase = s * RPT
    for t in range(RPT // CCH):
        pltpu.sync_copy(msg0, acc.at[pl.ds(rbase + t * CCH, CCH)])
    plsc.subcore_barrier()

    pltpu.sync_copy(wg2_hbm, wg2v)
    pltpu.sync_copy(bg2_hbm, bg2v)
    wg2k = [wg2v[pl.ds(16 * k, 16)] for k in range(4)]
    bg2 = bg2v[...]
    lane = lax.iota(jnp.int32, 16)
    gdn = lax.GatherDimensionNumbers(
        offset_dims=(), collapsed_slice_dims=(0,), start_index_map=(0,))
    perms = [jnp.bitwise_and(lane + sh, 15)[:, None] for sh in (8, 4, 2, 1)]

    ebase = wid * EPW

    def issue(ci, b):
        cb = ebase + ci * CCH
        pltpu.sync_copy(src_hbm.at[pl.ds(cb, CCH)], idx_s[b])
        pltpu.sync_copy(dst_hbm.at[pl.ds(cb, CCH)], idx_d[b])
        pltpu.async_copy(ts_hbm.at[idx_s[b]], gs[b], semg[b])
        pltpu.async_copy(td_hbm.at[idx_d[b]], gd[b], semg[b])

    def wait_gathers(b):
        pltpu.make_async_copy(ts_hbm.at[idx_s[b]], gs[b], semg[b]).wait()
        pltpu.make_async_copy(td_hbm.at[idx_d[b]], gd[b], semg[b]).wait()

    def wait_scatter(b):
        pltpu.make_async_copy(msg[b], acc.at[sidx[b]], semc[b]).wait()

    def copy_idx(b):
        sidx[b][pl.ds(0, 16)] = idx_d[b][pl.ds(0, 16)]
        sidx[b][pl.ds(16, 16)] = idx_d[b][pl.ds(16, 16)]
        sidx[b][pl.ds(24, 16)] = idx_d[b][pl.ds(24, 16)]

    def edge_one(e2, b):
        gh_acc = None
        for k in range(4):
            ghk = (gs[b][e2, pl.ds(128 + 16 * k, 16)]
                   + gd[b][e2, pl.ds(128 + 16 * k, 16)]
                   + pe[e2, pl.ds(128 + 16 * k, 16)])
            ghk = jnp.maximum(ghk, 0.0) * wg2k[k]
            gh_acc = ghk if gh_acc is None else gh_acc + ghk
        v = gh_acc
        for pm in perms:
            v = v + lax.gather(
                v, pm, gdn, slice_sizes=(1,),
                mode=lax.GatherScatterMode.PROMISE_IN_BOUNDS)
        tv = v + bg2
        gate = 1.0 / (1.0 + jnp.exp(-tv))
        for k in range(8):
            raw = (gs[b][e2, pl.ds(16 * k, 16)]
                   + gd[b][e2, pl.ds(16 * k, 16)]
                   + pe[e2, pl.ds(16 * k, 16)])
            msg[b][e2, pl.ds(16 * k, 16)] = jnp.maximum(raw, 0.0) * gate

    def halfstep(i, ci, b):
        wait_gathers(b)
        pltpu.sync_copy(pe_hbm.at[pl.ds(ebase + ci * CCH, CCH)], pe)

        @pl.when(i > 0)
        def _():
            wait_scatter(b)

        @plsc.parallel_loop(0, CCH, 1, unroll=4)
        def _(e2):
            edge_one(e2, b)

        copy_idx(b)
        pltpu.async_copy(msg[b], acc.at[sidx[b]], semc[b], add=True)

        @pl.when(i < NCCH // 2 - 1)
        def _():
            issue(ci + 2, b)

    issue(0, 0)
    issue(1, 1)

    def chunk_pair(i, carry):
        halfstep(i, 2 * i, 0)
        halfstep(i, 2 * i + 1, 1)
        return carry

    lax.fori_loop(0, NCCH // 2, chunk_pair, 0)
    wait_scatter(0)
    wait_scatter(1)
    plsc.subcore_barrier()
    pltpu.sync_copy(acc.at[pl.ds(rbase, RPT)], out_hbm.at[c, pl.ds(rbase, RPT)])


_conv_sc = pl.kernel(
    _conv_sc_body,
    out_type=jax.ShapeDtypeStruct((NCORE, NP, 128), jnp.float32),
    mesh=_MESH,
    scratch_types=[
        pltpu.VMEM((CCH,), jnp.int32),
        pltpu.VMEM((CCH,), jnp.int32),
        pltpu.VMEM((CCH,), jnp.int32),
        pltpu.VMEM((CCH,), jnp.int32),
        pltpu.VMEM((CCH,), jnp.int32),
        pltpu.VMEM((CCH,), jnp.int32),
        pltpu.VMEM((CCH, 192), jnp.float32),
        pltpu.VMEM((CCH, 192), jnp.float32),
        pltpu.VMEM((CCH, 192), jnp.float32),
        pltpu.VMEM((CCH, 192), jnp.float32),
        pltpu.VMEM((CCH, 192), jnp.float32),
        pltpu.VMEM((CCH, 128), jnp.float32),
        pltpu.VMEM((CCH, 128), jnp.float32),
        pltpu.VMEM((64,), jnp.float32),
        pltpu.VMEM((16,), jnp.float32),
        pltpu.VMEM_SHARED((NP, 128), jnp.float32),
        pltpu.SemaphoreType.DMA,
        pltpu.SemaphoreType.DMA,
        pltpu.SemaphoreType.DMA,
        pltpu.SemaphoreType.DMA,
    ],
    compiler_params=_SC_PARAMS,
)


def _count_sc_body(dst_hbm, out_hbm, idx_d, ones, acc, sem0):
    c = lax.axis_index("c")
    s = lax.axis_index("s")
    wid = c * NSUB + s

    zero16 = jnp.zeros((16,), jnp.float32)

    def zrow(i, carry):
        ones[i, pl.ds(0, 16)] = zero16
        return carry

    lax.fori_loop(0, CH, zrow, 0)
    rbase = s * RPT
    for t in range(RPT // CH):
        pltpu.sync_copy(ones, acc.at[pl.ds(rbase + t * CH, CH)])
    plsc.subcore_barrier()

    lane = lax.iota(jnp.int32, 16)
    cnt_vec = jnp.where(lane == 0, 1.0, 0.0).astype(jnp.float32)

    def orow(i, carry):
        ones[i, pl.ds(0, 16)] = cnt_vec
        return carry

    lax.fori_loop(0, CH, orow, 0)

    ebase = wid * EPW

    def chunk_body(i, carry):
        cb = ebase + i * CH
        pltpu.sync_copy(dst_hbm.at[pl.ds(cb, CH)], idx_d)
        pltpu.sync_copy(ones, acc.at[idx_d], add=True)
        return carry

    lax.fori_loop(0, NCHUNK, chunk_body, 0)
    plsc.subcore_barrier()
    pltpu.sync_copy(acc.at[pl.ds(rbase, RPT)], out_hbm.at[c, pl.ds(rbase, RPT)])


_count_sc = pl.kernel(
    _count_sc_body,
    out_type=jax.ShapeDtypeStruct((NCORE, NP, 16), jnp.float32),
    mesh=_MESH,
    scratch_types=[
        pltpu.VMEM((CH,), jnp.int32),
        pltpu.VMEM((CH, 16), jnp.float32),
        pltpu.VMEM_SHARED((NP, 16), jnp.float32),
        pltpu.SemaphoreType.DMA,
    ],
    compiler_params=_SC_PARAMS,
)


def _clf_sc_body(ht_hbm, src_hbm, dst_hbm, zs_hbm, zd_hbm,
                 idx_s0, idx_d0, idx_s1, idx_d1, idx_s2, idx_d2,
                 idx_s3, idx_d3, idx_s4, idx_d4,
                 u0, v0, u1, v1, u2, v2, u3, v3, u4, v4,
                 semg0, semg1, semg2, semg3, semg4,
                 semw0, semw1, semw2, semw3, semw4):
    c = lax.axis_index("c")
    s = lax.axis_index("s")
    wid = c * NSUB + s
    ebase = wid * EPW

    idx_s = [idx_s0, idx_s1, idx_s2, idx_s3, idx_s4]
    idx_d = [idx_d0, idx_d1, idx_d2, idx_d3, idx_d4]
    u = [u0, u1, u2, u3, u4]
    v = [v0, v1, v2, v3, v4]
    semg = [semg0, semg1, semg2, semg3, semg4]
    semw = [semw0, semw1, semw2, semw3, semw4]

    def issue(ci, b):
        cb = ebase + ci * CH
        pltpu.sync_copy(src_hbm.at[pl.ds(cb, CH)], idx_s[b])
        pltpu.sync_copy(dst_hbm.at[pl.ds(cb, CH)], idx_d[b])
        pltpu.async_copy(ht_hbm.at[idx_s[b]], u[b], semg[b])
        pltpu.async_copy(ht_hbm.at[idx_d[b]], v[b], semg[b])

    def wait_gathers(b):
        pltpu.make_async_copy(ht_hbm.at[idx_s[b]], u[b], semg[b]).wait()
        pltpu.make_async_copy(ht_hbm.at[idx_d[b]], v[b], semg[b]).wait()

    def start_writes(ci, b):
        cb = ebase + ci * CH
        pltpu.async_copy(u[b], zs_hbm.at[pl.ds(cb, CH)], semw[b])
        pltpu.async_copy(v[b], zd_hbm.at[pl.ds(cb, CH)], semw[b])

    def wait_writes(ci, b):
        cb = ebase + ci * CH
        pltpu.make_async_copy(u[b], zs_hbm.at[pl.ds(cb, CH)], semw[b]).wait()
        pltpu.make_async_copy(v[b], zd_hbm.at[pl.ds(cb, CH)], semw[b]).wait()

    for b in range(NSLOT):
        issue(b, b)

    nq = NCHUNK // NSLOT

    def quint(i, carry):
        for b in range(NSLOT):
            wait_gathers(b)
            start_writes(i * NSLOT + b, b)
        for b in range(NSLOT):
            @pl.when(i < nq - 1)
            def _(b=b):
                wait_writes(i * NSLOT + b, b)
                issue((i + 1) * NSLOT + b, b)
        return carry

    lax.fori_loop(0, nq, quint, 0)
    for b in range(NSLOT):
        wait_writes((nq - 1) * NSLOT + b, b)


_clf_sc = pl.kernel(
    _clf_sc_body,
    out_type=(jax.ShapeDtypeStruct((E, 128), jnp.float32),
              jax.ShapeDtypeStruct((E, 128), jnp.float32)),
    mesh=_MESH,
    scratch_types=(
        [pltpu.VMEM((CH,), jnp.int32) for _ in range(2 * NSLOT)]
        + [pltpu.VMEM((CH, 128), jnp.float32) for _ in range(2 * NSLOT)]
        + [pltpu.SemaphoreType.DMA for _ in range(2 * NSLOT)]
    ),
    compiler_params=_SC_PARAMS,
)


def _node_pre_body(x_ref, w1_ref, w2_ref, w3_ref, b3_ref,
                   ts_ref, td_ref, self_ref):
    x = x_ref[...]
    ts_ref[...] = jnp.dot(x, w1_ref[...], preferred_element_type=jnp.float32)
    td_ref[...] = jnp.dot(x, w2_ref[...], preferred_element_type=jnp.float32)
    self_ref[...] = (
        jnp.dot(x, w3_ref[...], preferred_element_type=jnp.float32)
        + b3_ref[...])


def _node_pre(x, w1, w2, w3, b3):
    return pl.pallas_call(
        _node_pre_body,
        grid=(NP // BN,),
        in_specs=[
            pl.BlockSpec((BN, 128), lambda i: (i, 0)),
            pl.BlockSpec((128, 192), lambda i: (0, 0)),
            pl.BlockSpec((128, 192), lambda i: (0, 0)),
            pl.BlockSpec((128, 128), lambda i: (0, 0)),
            pl.BlockSpec((1, 128), lambda i: (0, 0)),
        ],
        out_specs=[
            pl.BlockSpec((BN, 192), lambda i: (i, 0)),
            pl.BlockSpec((BN, 192), lambda i: (i, 0)),
            pl.BlockSpec((BN, 128), lambda i: (i, 0)),
        ],
        out_shape=[
            jax.ShapeDtypeStruct((NP, 192), jnp.float32),
            jax.ShapeDtypeStruct((NP, 192), jnp.float32),
            jax.ShapeDtypeStruct((NP, 128), jnp.float32),
        ],
    )(x, w1, w2, w3, b3)


def _edge_pre_body(e_ref, w0_ref, b0_ref, w1_ref, b1_ref,
                   p0_ref, p1_ref):
    e = e_ref[...]
    p0_ref[...] = (
        jnp.dot(e, w0_ref[...], preferred_element_type=jnp.float32)
        + b0_ref[...])
    p1_ref[...] = (
        jnp.dot(e, w1_ref[...], preferred_element_type=jnp.float32)
        + b1_ref[...])


def _edge_pre(e, w0, b0, w1, b1):
    return pl.pallas_call(
        _edge_pre_body,
        grid=(E // BEDGE,),
        in_specs=[
            pl.BlockSpec((BEDGE, 16), lambda i: (i, 0)),
            pl.BlockSpec((16, 192), lambda i: (0, 0)),
            pl.BlockSpec((1, 192), lambda i: (0, 0)),
            pl.BlockSpec((16, 192), lambda i: (0, 0)),
            pl.BlockSpec((1, 192), lambda i: (0, 0)),
        ],
        out_specs=[
            pl.BlockSpec((BEDGE, 192), lambda i: (i, 0)),
            pl.BlockSpec((BEDGE, 192), lambda i: (i, 0)),
        ],
        out_shape=[
            jax.ShapeDtypeStruct((E, 192), jnp.float32),
            jax.ShapeDtypeStruct((E, 192), jnp.float32),
        ],
    )(e, w0, b0, w1, b1)


def _finish_node(a0, a1, c0, c1, slf, g, b):
    sums = a0 + a1
    cnt = jnp.sum(c0 + c1, axis=1, keepdims=True)
    agg = sums / jnp.maximum(cnt, 1.0)
    o = agg + slf
    m = jnp.mean(o, axis=-1, keepdims=True)
    v = jnp.mean((o - m) ** 2, axis=-1, keepdims=True)
    hn = (o - m) / jnp.sqrt(v + 1e-5) * g + b
    return jnp.maximum(hn, 0.0)


def _combine0_body(a0_ref, a1_ref, c0_ref, c1_ref, self_ref, g_ref, b_ref,
                   w1_ref, w2_ref, w3_ref, b3_ref,
                   ts_ref, td_ref, self1_ref):
    h = _finish_node(a0_ref[...], a1_ref[...], c0_ref[...], c1_ref[...],
                     self_ref[...], g_ref[...], b_ref[...])
    ts_ref[...] = jnp.dot(h, w1_ref[...], preferred_element_type=jnp.float32)
    td_ref[...] = jnp.dot(h, w2_ref[...], preferred_element_type=jnp.float32)
    self1_ref[...] = (
        jnp.dot(h, w3_ref[...], preferred_element_type=jnp.float32)
        + b3_ref[...])


def _combine0(a0, a1, c0, c1, slf, g, b, w1, w2, w3, b3):
    return pl.pallas_call(
        _combine0_body,
        grid=(NP // BN,),
        in_specs=[
            pl.BlockSpec((BN, 128), lambda i: (i, 0)),
            pl.BlockSpec((BN, 128), lambda i: (i, 0)),
            pl.BlockSpec((BN, 16), lambda i: (i, 0)),
            pl.BlockSpec((BN, 16), lambda i: (i, 0)),
            pl.BlockSpec((BN, 128), lambda i: (i, 0)),
            pl.BlockSpec((1, 128), lambda i: (0, 0)),
            pl.BlockSpec((1, 128), lambda i: (0, 0)),
            pl.BlockSpec((128, 192), lambda i: (0, 0)),
            pl.BlockSpec((128, 192), lambda i: (0, 0)),
            pl.BlockSpec((128, 128), lambda i: (0, 0)),
            pl.BlockSpec((1, 128), lambda i: (0, 0)),
        ],
        out_specs=[
            pl.BlockSpec((BN, 192), lambda i: (i, 0)),
            pl.BlockSpec((BN, 192), lambda i: (i, 0)),
            pl.BlockSpec((BN, 128), lambda i: (i, 0)),
        ],
        out_shape=[
            jax.ShapeDtypeStruct((NP, 192), jnp.float32),
            jax.ShapeDtypeStruct((NP, 192), jnp.float32),
            jax.ShapeDtypeStruct((NP, 128), jnp.float32),
        ],
    )(a0, a1, c0, c1, slf, g, b, w1, w2, w3, b3)


def _combine1_body(a0_ref, a1_ref, c0_ref, c1_ref, self_ref, g_ref, b_ref,
                   hp_ref, wih_ref, bih_ref, whh_ref, bhh_ref,
                   ht_ref):
    h = _finish_node(a0_ref[...], a1_ref[...], c0_ref[...], c1_ref[...],
                     self_ref[...], g_ref[...], b_ref[...])
    hp = hp_ref[...]
    gi = jnp.dot(h, wih_ref[...], preferred_element_type=jnp.float32) \
        + bih_ref[...]
    gh = jnp.dot(hp, whh_ref[...], preferred_element_type=jnp.float32) \
        + bhh_ref[...]
    r = jax.nn.sigmoid(gi[:, :128] + gh[:, :128])
    z = jax.nn.sigmoid(gi[:, 128:256] + gh[:, 128:256])
    n = jnp.tanh(gi[:, 256:384] + r * gh[:, 256:384])
    ht_ref[...] = (1.0 - z) * n + z * hp


def _combine1(a0, a1, c0, c1, slf, g, b, hp, wih, bih, whh, bhh):
    return pl.pallas_call(
        _combine1_body,
        grid=(NP // BN,),
        in_specs=[
            pl.BlockSpec((BN, 128), lambda i: (i, 0)),
            pl.BlockSpec((BN, 128), lambda i: (i, 0)),
            pl.BlockSpec((BN, 16), lambda i: (i, 0)),
            pl.BlockSpec((BN, 16), lambda i: (i, 0)),
            pl.BlockSpec((BN, 128), lambda i: (i, 0)),
            pl.BlockSpec((1, 128), lambda i: (0, 0)),
            pl.BlockSpec((1, 128), lambda i: (0, 0)),
            pl.BlockSpec((BN, 128), lambda i: (i, 0)),
            pl.BlockSpec((128, 384), lambda i: (0, 0)),
            pl.BlockSpec((1, 384), lambda i: (0, 0)),
            pl.BlockSpec((128, 384), lambda i: (0, 0)),
            pl.BlockSpec((1, 384), lambda i: (0, 0)),
        ],
        out_specs=pl.BlockSpec((BN, 128), lambda i: (i, 0)),
        out_shape=jax.ShapeDtypeStruct((NP, 128), jnp.float32),
    )(a0, a1, c0, c1, slf, g, b, hp, wih, bih, whh, bhh)


def _head_body(zs_ref, zd_ref, e_ref, wem_ref, bm1_ref,
               ws_ref, wd_ref, wa_ref, wp_ref,
               wm2_ref, bm2_ref, out_ref):
    zs = zs_ref[...]
    zd = zd_ref[...]
    hm = (jnp.dot(e_ref[...], wem_ref[...],
                  preferred_element_type=jnp.float32)
          + bm1_ref[...])
    hm = hm + jnp.dot(zs, ws_ref[...], preferred_element_type=jnp.float32)
    hm = hm + jnp.dot(zd, wd_ref[...], preferred_element_type=jnp.float32)
    hm = hm + jnp.dot(jnp.abs(zs - zd), wa_ref[...],
                      preferred_element_type=jnp.float32)
    hm = hm + jnp.dot(zs * zd, wp_ref[...],
                      preferred_element_type=jnp.float32)
    hm = jnp.maximum(hm, 0.0)
    out_ref[...] = (
        jnp.dot(hm, wm2_ref[...], preferred_element_type=jnp.float32)
        + bm2_ref[...])


def _head(zs, zd, e, wem_t, bm1, ws_t, wd_t, wa_t, wp_t, wm2_t, bm2):
    return pl.pallas_call(
        _head_body,
        grid=(E // BEDGE,),
        in_specs=[
            pl.BlockSpec((BEDGE, 128), lambda i: (i, 0)),
            pl.BlockSpec((BEDGE, 128), lambda i: (i, 0)),
            pl.BlockSpec((BEDGE, 16), lambda i: (i, 0)),
            pl.BlockSpec((16, 128), lambda i: (0, 0)),
            pl.BlockSpec((1, 128), lambda i: (0, 0)),
            pl.BlockSpec((128, 128), lambda i: (0, 0)),
            pl.BlockSpec((128, 128), lambda i: (0, 0)),
            pl.BlockSpec((128, 128), lambda i: (0, 0)),
            pl.BlockSpec((128, 128), lambda i: (0, 0)),
            pl.BlockSpec((128, 2), lambda i: (0, 0)),
            pl.BlockSpec((1, 2), lambda i: (0, 0)),
        ],
        out_specs=pl.BlockSpec((BEDGE, 2), lambda i: (i, 0)),
        out_shape=jax.ShapeDtypeStruct((E, 2), jnp.float32),
    )(zs, zd, e, wem_t, bm1, ws_t, wd_t, wa_t, wp_t, wm2_t, bm2)


def kernel(x, edge_index, edge_attr, h_prev, params):
    p = params
    src = edge_index[0]
    dst = edge_index[1]
    e = edge_attr

    x_pad = jnp.zeros((NP, D), jnp.float32).at[:N].set(x)
    hp_pad = jnp.zeros((NP, H), jnp.float32).at[:N].set(h_prev)

    wg1_0, wg1_1 = p['Wg1_0'], p['Wg1_1']
    w_p0 = jnp.concatenate([p['W_edge0'].T, wg1_0[:, 2 * D:].T], axis=1)
    b_p0 = jnp.concatenate([jnp.zeros((128,), jnp.float32), p['bg1_0']])
    w_p1 = jnp.concatenate([p['W_edge1'].T, wg1_1[:, 2 * D:].T], axis=1)
    b_p1 = jnp.concatenate([jnp.zeros((128,), jnp.float32), p['bg1_1']])
    wm1 = p['Wm1']
    pe0, pe1 = _edge_pre(e, w_p0, b_p0.reshape(1, -1),
                         w_p1, b_p1.reshape(1, -1))

    cnt = _count_sc(dst)

    w1 = jnp.concatenate([p['W_src0'].T, wg1_0[:, D:2 * D].T], axis=1)
    w2 = jnp.concatenate([p['W_dst0'].T, wg1_0[:, :D].T], axis=1)
    ts0, td0, slf0 = _node_pre(x_pad, w1, w2, p['W_self0'].T,
                               p['b_self0'].reshape(1, -1))

    wg2_0 = p['Wg2_0'].reshape(64)
    bg2_0 = jnp.broadcast_to(p['bg2_0'].reshape(1), (16,))
    acc0 = _conv_sc(ts0, td0, pe0, src, dst, wg2_0, bg2_0)

    w1b = jnp.concatenate([p['W_src1'].T, wg1_1[:, D:2 * D].T], axis=1)
    w2b = jnp.concatenate([p['W_dst1'].T, wg1_1[:, :D].T], axis=1)
    ts1, td1, slf1 = _combine0(acc0[0], acc0[1], cnt[0], cnt[1], slf0,
                               p['ln_g0'].reshape(1, -1),
                               p['ln_b0'].reshape(1, -1),
                               w1b, w2b, p['W_self1'].T,
                               p['b_self1'].reshape(1, -1))

    wg2_1 = p['Wg2_1'].reshape(64)
    bg2_1 = jnp.broadcast_to(p['bg2_1'].reshape(1), (16,))
    acc1 = _conv_sc(ts1, td1, pe1, src, dst, wg2_1, bg2_1)

    ht_pad = _combine1(acc1[0], acc1[1], cnt[0], cnt[1], slf1,
                       p['ln_g1'].reshape(1, -1),
                       p['ln_b1'].reshape(1, -1),
                       hp_pad,
                       p['W_ih'].T, p['b_ih'].reshape(1, -1),
                       p['W_hh'].T, p['b_hh'].reshape(1, -1))

    zs, zd = _clf_sc(ht_pad, src, dst)

    logits = _head(zs, zd, e,
                   wm1[:, 2 * H:2 * H + DE].T, p['bm1'].reshape(1, -1),
                   wm1[:, :H].T, wm1[:, H:2 * H].T,
                   wm1[:, 2 * H + DE:3 * H + DE].T, wm1[:, 3 * H + DE:].T,
                   p['Wm2'].T, p['bm2'].reshape(1, 2))
    return logits, ht_pad[:N]


# speedup vs baseline: 1.1442x; 1.1245x over previous
import functools

import jax
import jax.numpy as jnp
from jax import lax
from jax.experimental import pallas as pl
from jax.experimental.pallas import tpu as pltpu
from jax.experimental.pallas import tpu_sc as plsc

N = 10000
E = 320000
D = 128
DE = 16
H = 128

NP = 10240
NCORE = 2
NSUB = 16
NW = NCORE * NSUB
EPW = E // NW
CCH = 16
NCCH = EPW // CCH
CH = 80
NCHUNK = EPW // CH
NSLOT = 5
RPT = NP // NSUB

BN = 512
BEDGE = 2000

_MESH = plsc.VectorSubcoreMesh(core_axis_name="c", subcore_axis_name="s")
_SC_PARAMS = pltpu.CompilerParams(use_tc_tiling_on_sc=False)


def _conv_sc_body(ts_hbm, td_hbm, gt_hbm, ew_hbm, ge_hbm, src_hbm, dst_hbm,
                  wg2_hbm, bg2_hbm,
                  out_hbm,
                  idx_s0, idx_d0, idx_s1, idx_d1, sidx0, sidx1,
                  gs0, gd0, gj0, gi0, gs1, gd1, gj1, gi1,
                  ew0, ew1, ge0, ge1, msg0, msg1, wg2v, bg2v, acc,
                  semg0, semg1, semc0, semc1):
    c = lax.axis_index("c")
    s = lax.axis_index("s")
    wid = c * NSUB + s

    idx_s = [idx_s0, idx_s1]
    idx_d = [idx_d0, idx_d1]
    sidx = [sidx0, sidx1]
    gs = [gs0, gs1]
    gd = [gd0, gd1]
    gj = [gj0, gj1]
    gi = [gi0, gi1]
    ew = [ew0, ew1]
    ge = [ge0, ge1]
    msg = [msg0, msg1]
    semg = [semg0, semg1]
    semc = [semc0, semc1]

    zero16 = jnp.zeros((16,), jnp.float32)

    def zrow(i, carry):
        for k in range(8):
            msg0[i, pl.ds(16 * k, 16)] = zero16
        return carry

    lax.fori_loop(0, CCH, zrow, 0)
    rbase = s * RPT
    for t in range(RPT // CCH):
        pltpu.sync_copy(msg0, acc.at[pl.ds(rbase + t * CCH, CCH)])
    plsc.subcore_barrier()

    pltpu.sync_copy(wg2_hbm, wg2v)
    pltpu.sync_copy(bg2_hbm, bg2v)
    wg2k = [wg2v[pl.ds(16 * k, 16)] for k in range(4)]
    bg2 = bg2v[...]
    lane = lax.iota(jnp.int32, 16)
    gdn = lax.GatherDimensionNumbers(
        offset_dims=(), collapsed_slice_dims=(0,), start_index_map=(0,))
    perms = [jnp.bitwise_and(lane + sh, 15)[:, None] for sh in (8, 4, 2, 1)]

    ebase = wid * EPW

    def issue(ci, b):
        cb = ebase + ci * CCH
        pltpu.sync_copy(src_hbm.at[pl.ds(cb, CCH)], idx_s[b])
        pltpu.sync_copy(dst_hbm.at[pl.ds(cb, CCH)], idx_d[b])
        pltpu.async_copy(ts_hbm.at[idx_s[b]], gs[b], semg[b])
        pltpu.async_copy(td_hbm.at[idx_d[b]], gd[b], semg[b])
        pltpu.async_copy(gt_hbm.at[idx_s[b]], gj[b], semg[b])
        pltpu.async_copy(gt_hbm.at[idx_d[b]], gi[b], semg[b])
        pltpu.async_copy(ew_hbm.at[pl.ds(cb, CCH)], ew[b], semg[b])
        pltpu.async_copy(ge_hbm.at[pl.ds(cb, CCH)], ge[b], semg[b])

    def wait_gathers(b):
        pltpu.make_async_copy(ts_hbm.at[idx_s[b]], gs[b], semg[b]).wait()
        pltpu.make_async_copy(td_hbm.at[idx_d[b]], gd[b], semg[b]).wait()
        pltpu.make_async_copy(gt_hbm.at[idx_s[b]], gj[b], semg[b]).wait()
        pltpu.make_async_copy(gt_hbm.at[idx_d[b]], gi[b], semg[b]).wait()
        pltpu.make_async_copy(ew_hbm.at[pl.ds(0, CCH)], ew[b], semg[b]).wait()
        pltpu.make_async_copy(ge_hbm.at[pl.ds(0, CCH)], ge[b], semg[b]).wait()

    def wait_scatter(b):
        pltpu.make_async_copy(msg[b], acc.at[sidx[b]], semc[b]).wait()

    def copy_idx(b):
        sidx[b][pl.ds(0, 16)] = idx_d[b][pl.ds(0, 16)]

    def edge_one(e2, b):
        gh_acc = None
        for k in range(4):
            ghk = (gj[b][e2, pl.ds(16 * k, 16)]
                   + gi[b][e2, pl.ds(64 + 16 * k, 16)]
                   + ge[b][e2, pl.ds(16 * k, 16)])
            ghk = jnp.maximum(ghk, 0.0) * wg2k[k]
            gh_acc = ghk if gh_acc is None else gh_acc + ghk
        v = gh_acc
        for pm in perms:
            v = v + lax.gather(
                v, pm, gdn, slice_sizes=(1,),
                mode=lax.GatherScatterMode.PROMISE_IN_BOUNDS)
        tv = v + bg2
        gate = 1.0 / (1.0 + jnp.exp(-tv))
        for k in range(8):
            raw = (gs[b][e2, pl.ds(16 * k, 16)]
                   + gd[b][e2, pl.ds(16 * k, 16)]
                   + ew[b][e2, pl.ds(16 * k, 16)])
            msg[b][e2, pl.ds(16 * k, 16)] = jnp.maximum(raw, 0.0) * gate

    def halfstep(i, ci, b, last):
        wait_gathers(b)

        @pl.when(i > 0)
        def _():
            wait_scatter(b)

        @plsc.parallel_loop(0, CCH, 1, unroll=4)
        def _(e2):
            edge_one(e2, b)

        copy_idx(b)
        pltpu.async_copy(msg[b], acc.at[sidx[b]], semc[b], add=True)

        @pl.when(i < last)
        def _():
            issue(ci + 2, b)

    issue(0, 0)
    issue(1, 1)

    npair = NCCH // 2

    def chunk_pair(i, carry):
        halfstep(i, 2 * i, 0, npair)
        halfstep(i, 2 * i + 1, 1, npair - 1)
        return carry

    lax.fori_loop(0, npair, chunk_pair, 0)
    halfstep(npair, NCCH - 1, 0, -1)
    wait_scatter(0)
    wait_scatter(1)
    plsc.subcore_barrier()
    pltpu.sync_copy(acc.at[pl.ds(rbase, RPT)], out_hbm.at[c, pl.ds(rbase, RPT)])


_conv_sc = pl.kernel(
    _conv_sc_body,
    out_type=jax.ShapeDtypeStruct((NCORE, NP, 128), jnp.float32),
    mesh=_MESH,
    scratch_types=(
        [pltpu.VMEM((CCH,), jnp.int32) for _ in range(6)]
        + [pltpu.VMEM((CCH, 128), jnp.float32) for _ in range(14)]
        + [pltpu.VMEM((64,), jnp.float32), pltpu.VMEM((16,), jnp.float32),
           pltpu.VMEM_SHARED((NP, 128), jnp.float32)]
        + [pltpu.SemaphoreType.DMA for _ in range(4)]
    ),
    compiler_params=_SC_PARAMS,
)


def _count_sc_body(dst_hbm, out_hbm, idx_d, ones, acc, sem0):
    c = lax.axis_index("c")
    s = lax.axis_index("s")
    wid = c * NSUB + s

    zero16 = jnp.zeros((16,), jnp.float32)

    def zrow(i, carry):
        ones[i, pl.ds(0, 16)] = zero16
        return carry

    lax.fori_loop(0, CH, zrow, 0)
    rbase = s * RPT
    for t in range(RPT // CH):
        pltpu.sync_copy(ones, acc.at[pl.ds(rbase + t * CH, CH)])
    plsc.subcore_barrier()

    lane = lax.iota(jnp.int32, 16)
    cnt_vec = jnp.where(lane == 0, 1.0, 0.0).astype(jnp.float32)

    def orow(i, carry):
        ones[i, pl.ds(0, 16)] = cnt_vec
        return carry

    lax.fori_loop(0, CH, orow, 0)

    ebase = wid * EPW

    def chunk_body(i, carry):
        cb = ebase + i * CH
        pltpu.sync_copy(dst_hbm.at[pl.ds(cb, CH)], idx_d)
        pltpu.sync_copy(ones, acc.at[idx_d], add=True)
        return carry

    lax.fori_loop(0, NCHUNK, chunk_body, 0)
    plsc.subcore_barrier()
    pltpu.sync_copy(acc.at[pl.ds(rbase, RPT)], out_hbm.at[c, pl.ds(rbase, RPT)])


_count_sc = pl.kernel(
    _count_sc_body,
    out_type=jax.ShapeDtypeStruct((NCORE, NP, 16), jnp.float32),
    mesh=_MESH,
    scratch_types=[
        pltpu.VMEM((CH,), jnp.int32),
        pltpu.VMEM((CH, 16), jnp.float32),
        pltpu.VMEM_SHARED((NP, 16), jnp.float32),
        pltpu.SemaphoreType.DMA,
    ],
    compiler_params=_SC_PARAMS,
)


def _clf_sc_body(ht_hbm, src_hbm, dst_hbm, zs_hbm, zd_hbm,
                 idx_s0, idx_d0, idx_s1, idx_d1, idx_s2, idx_d2,
                 idx_s3, idx_d3, idx_s4, idx_d4,
                 u0, v0, u1, v1, u2, v2, u3, v3, u4, v4,
                 semg0, semg1, semg2, semg3, semg4,
                 semw0, semw1, semw2, semw3, semw4):
    c = lax.axis_index("c")
    s = lax.axis_index("s")
    wid = c * NSUB + s
    ebase = wid * EPW

    idx_s = [idx_s0, idx_s1, idx_s2, idx_s3, idx_s4]
    idx_d = [idx_d0, idx_d1, idx_d2, idx_d3, idx_d4]
    u = [u0, u1, u2, u3, u4]
    v = [v0, v1, v2, v3, v4]
    semg = [semg0, semg1, semg2, semg3, semg4]
    semw = [semw0, semw1, semw2, semw3, semw4]

    def issue(ci, b):
        cb = ebase + ci * CH
        pltpu.sync_copy(src_hbm.at[pl.ds(cb, CH)], idx_s[b])
        pltpu.sync_copy(dst_hbm.at[pl.ds(cb, CH)], idx_d[b])
        pltpu.async_copy(ht_hbm.at[idx_s[b]], u[b], semg[b])
        pltpu.async_copy(ht_hbm.at[idx_d[b]], v[b], semg[b])

    def wait_gathers(b):
        pltpu.make_async_copy(ht_hbm.at[idx_s[b]], u[b], semg[b]).wait()
        pltpu.make_async_copy(ht_hbm.at[idx_d[b]], v[b], semg[b]).wait()

    def start_writes(ci, b):
        cb = ebase + ci * CH
        pltpu.async_copy(u[b], zs_hbm.at[pl.ds(cb, CH)], semw[b])
        pltpu.async_copy(v[b], zd_hbm.at[pl.ds(cb, CH)], semw[b])

    def wait_writes(ci, b):
        cb = ebase + ci * CH
        pltpu.make_async_copy(u[b], zs_hbm.at[pl.ds(cb, CH)], semw[b]).wait()
        pltpu.make_async_copy(v[b], zd_hbm.at[pl.ds(cb, CH)], semw[b]).wait()

    for b in range(NSLOT):
        issue(b, b)

    nq = NCHUNK // NSLOT

    def quint(i, carry):
        for b in range(NSLOT):
            wait_gathers(b)
            start_writes(i * NSLOT + b, b)
        for b in range(NSLOT):
            @pl.when(i < nq - 1)
            def _(b=b):
                wait_writes(i * NSLOT + b, b)
                issue((i + 1) * NSLOT + b, b)
        return carry

    lax.fori_loop(0, nq, quint, 0)
    for b in range(NSLOT):
        wait_writes((nq - 1) * NSLOT + b, b)


_clf_sc = pl.kernel(
    _clf_sc_body,
    out_type=(jax.ShapeDtypeStruct((E, 128), jnp.float32),
              jax.ShapeDtypeStruct((E, 128), jnp.float32)),
    mesh=_MESH,
    scratch_types=(
        [pltpu.VMEM((CH,), jnp.int32) for _ in range(2 * NSLOT)]
        + [pltpu.VMEM((CH, 128), jnp.float32) for _ in range(2 * NSLOT)]
        + [pltpu.SemaphoreType.DMA for _ in range(2 * NSLOT)]
    ),
    compiler_params=_SC_PARAMS,
)


def _node_pre_body(x_ref, w1_ref, w2_ref, wg_ref, w3_ref, b3_ref,
                   ts_ref, td_ref, gt_ref, self_ref):
    x = x_ref[...]
    ts_ref[...] = jnp.dot(x, w1_ref[...], preferred_element_type=jnp.float32)
    td_ref[...] = jnp.dot(x, w2_ref[...], preferred_element_type=jnp.float32)
    gt_ref[...] = jnp.dot(x, wg_ref[...], preferred_element_type=jnp.float32)
    self_ref[...] = (
        jnp.dot(x, w3_ref[...], preferred_element_type=jnp.float32)
        + b3_ref[...])


def _node_pre(x, w1, w2, wg, w3, b3):
    return pl.pallas_call(
        _node_pre_body,
        grid=(NP // BN,),
        in_specs=[
            pl.BlockSpec((BN, 128), lambda i: (i, 0)),
            pl.BlockSpec((128, 128), lambda i: (0, 0)),
            pl.BlockSpec((128, 128), lambda i: (0, 0)),
            pl.BlockSpec((128, 128), lambda i: (0, 0)),
            pl.BlockSpec((128, 128), lambda i: (0, 0)),
            pl.BlockSpec((1, 128), lambda i: (0, 0)),
        ],
        out_specs=[
            pl.BlockSpec((BN, 128), lambda i: (i, 0)),
            pl.BlockSpec((BN, 128), lambda i: (i, 0)),
            pl.BlockSpec((BN, 128), lambda i: (i, 0)),
            pl.BlockSpec((BN, 128), lambda i: (i, 0)),
        ],
        out_shape=[
            jax.ShapeDtypeStruct((NP, 128), jnp.float32),
            jax.ShapeDtypeStruct((NP, 128), jnp.float32),
            jax.ShapeDtypeStruct((NP, 128), jnp.float32),
            jax.ShapeDtypeStruct((NP, 128), jnp.float32),
        ],
    )(x, w1, w2, wg, w3, b3)


def _edge_pre_body(e_ref, we0_ref, wg0_ref, bg0_ref, we1_ref, wg1_ref,
                   bg1_ref, ew0_ref, ge0_ref, ew1_ref, ge1_ref):
    e = e_ref[...]
    ew0_ref[...] = jnp.dot(e, we0_ref[...],
                           preferred_element_type=jnp.float32)
    ge0_ref[...] = (jnp.dot(e, wg0_ref[...],
                            preferred_element_type=jnp.float32)
                    + bg0_ref[...])
    ew1_ref[...] = jnp.dot(e, we1_ref[...],
                           preferred_element_type=jnp.float32)
    ge1_ref[...] = (jnp.dot(e, wg1_ref[...],
                            preferred_element_type=jnp.float32)
                    + bg1_ref[...])


def _edge_pre(e, we0, wg0, bg0, we1, wg1, bg1):
    return pl.pallas_call(
        _edge_pre_body,
        grid=(E // BEDGE,),
        in_specs=[
            pl.BlockSpec((BEDGE, 16), lambda i: (i, 0)),
            pl.BlockSpec((16, 128), lambda i: (0, 0)),
            pl.BlockSpec((16, 128), lambda i: (0, 0)),
            pl.BlockSpec((1, 128), lambda i: (0, 0)),
            pl.BlockSpec((16, 128), lambda i: (0, 0)),
            pl.BlockSpec((16, 128), lambda i: (0, 0)),
            pl.BlockSpec((1, 128), lambda i: (0, 0)),
        ],
        out_specs=[
            pl.BlockSpec((BEDGE, 128), lambda i: (i, 0)),
            pl.BlockSpec((BEDGE, 128), lambda i: (i, 0)),
            pl.BlockSpec((BEDGE, 128), lambda i: (i, 0)),
            pl.BlockSpec((BEDGE, 128), lambda i: (i, 0)),
        ],
        out_shape=[
            jax.ShapeDtypeStruct((E, 128), jnp.float32),
            jax.ShapeDtypeStruct((E, 128), jnp.float32),
            jax.ShapeDtypeStruct((E, 128), jnp.float32),
            jax.ShapeDtypeStruct((E, 128), jnp.float32),
        ],
    )(e, we0, wg0, bg0, we1, wg1, bg1)


def _finish_node(a0, a1, c0, c1, slf, g, b):
    sums = a0 + a1
    cnt = jnp.sum(c0 + c1, axis=1, keepdims=True)
    agg = sums / jnp.maximum(cnt, 1.0)
    o = agg + slf
    m = jnp.mean(o, axis=-1, keepdims=True)
    v = jnp.mean((o - m) ** 2, axis=-1, keepdims=True)
    hn = (o - m) / jnp.sqrt(v + 1e-5) * g + b
    return jnp.maximum(hn, 0.0)


def _combine0_body(a0_ref, a1_ref, c0_ref, c1_ref, self_ref, g_ref, b_ref,
                   w1_ref, w2_ref, wg_ref, w3_ref, b3_ref,
                   ts_ref, td_ref, gt_ref, self1_ref):
    h = _finish_node(a0_ref[...], a1_ref[...], c0_ref[...], c1_ref[...],
                     self_ref[...], g_ref[...], b_ref[...])
    ts_ref[...] = jnp.dot(h, w1_ref[...], preferred_element_type=jnp.float32)
    td_ref[...] = jnp.dot(h, w2_ref[...], preferred_element_type=jnp.float32)
    gt_ref[...] = jnp.dot(h, wg_ref[...], preferred_element_type=jnp.float32)
    self1_ref[...] = (
        jnp.dot(h, w3_ref[...], preferred_element_type=jnp.float32)
        + b3_ref[...])


def _combine0(a0, a1, c0, c1, slf, g, b, w1, w2, wg, w3, b3):
    return pl.pallas_call(
        _combine0_body,
        grid=(NP // BN,),
        in_specs=[
            pl.BlockSpec((BN, 128), lambda i: (i, 0)),
            pl.BlockSpec((BN, 128), lambda i: (i, 0)),
            pl.BlockSpec((BN, 16), lambda i: (i, 0)),
            pl.BlockSpec((BN, 16), lambda i: (i, 0)),
            pl.BlockSpec((BN, 128), lambda i: (i, 0)),
            pl.BlockSpec((1, 128), lambda i: (0, 0)),
            pl.BlockSpec((1, 128), lambda i: (0, 0)),
            pl.BlockSpec((128, 128), lambda i: (0, 0)),
            pl.BlockSpec((128, 128), lambda i: (0, 0)),
            pl.BlockSpec((128, 128), lambda i: (0, 0)),
            pl.BlockSpec((128, 128), lambda i: (0, 0)),
            pl.BlockSpec((1, 128), lambda i: (0, 0)),
        ],
        out_specs=[
            pl.BlockSpec((BN, 128), lambda i: (i, 0)),
            pl.BlockSpec((BN, 128), lambda i: (i, 0)),
            pl.BlockSpec((BN, 128), lambda i: (i, 0)),
            pl.BlockSpec((BN, 128), lambda i: (i, 0)),
        ],
        out_shape=[
            jax.ShapeDtypeStruct((NP, 128), jnp.float32),
            jax.ShapeDtypeStruct((NP, 128), jnp.float32),
            jax.ShapeDtypeStruct((NP, 128), jnp.float32),
            jax.ShapeDtypeStruct((NP, 128), jnp.float32),
        ],
    )(a0, a1, c0, c1, slf, g, b, w1, w2, wg, w3, b3)


def _combine1_body(a0_ref, a1_ref, c0_ref, c1_ref, self_ref, g_ref, b_ref,
                   hp_ref, wih_ref, bih_ref, whh_ref, bhh_ref,
                   ht_ref):
    h = _finish_node(a0_ref[...], a1_ref[...], c0_ref[...], c1_ref[...],
                     self_ref[...], g_ref[...], b_ref[...])
    hp = hp_ref[...]
    gi = jnp.dot(h, wih_ref[...], preferred_element_type=jnp.float32) \
        + bih_ref[...]
    gh = jnp.dot(hp, whh_ref[...], preferred_element_type=jnp.float32) \
        + bhh_ref[...]
    r = jax.nn.sigmoid(gi[:, :128] + gh[:, :128])
    z = jax.nn.sigmoid(gi[:, 128:256] + gh[:, 128:256])
    n = jnp.tanh(gi[:, 256:384] + r * gh[:, 256:384])
    ht_ref[...] = (1.0 - z) * n + z * hp


def _combine1(a0, a1, c0, c1, slf, g, b, hp, wih, bih, whh, bhh):
    return pl.pallas_call(
        _combine1_body,
        grid=(NP // BN,),
        in_specs=[
            pl.BlockSpec((BN, 128), lambda i: (i, 0)),
            pl.BlockSpec((BN, 128), lambda i: (i, 0)),
            pl.BlockSpec((BN, 16), lambda i: (i, 0)),
            pl.BlockSpec((BN, 16), lambda i: (i, 0)),
            pl.BlockSpec((BN, 128), lambda i: (i, 0)),
            pl.BlockSpec((1, 128), lambda i: (0, 0)),
            pl.BlockSpec((1, 128), lambda i: (0, 0)),
            pl.BlockSpec((BN, 128), lambda i: (i, 0)),
            pl.BlockSpec((128, 384), lambda i: (0, 0)),
            pl.BlockSpec((1, 384), lambda i: (0, 0)),
            pl.BlockSpec((128, 384), lambda i: (0, 0)),
            pl.BlockSpec((1, 384), lambda i: (0, 0)),
        ],
        out_specs=pl.BlockSpec((BN, 128), lambda i: (i, 0)),
        out_shape=jax.ShapeDtypeStruct((NP, 128), jnp.float32),
    )(a0, a1, c0, c1, slf, g, b, hp, wih, bih, whh, bhh)


def _head_body(zs_ref, zd_ref, e_ref, wem_ref, bm1_ref,
               ws_ref, wd_ref, wa_ref, wp_ref,
               wm2_ref, bm2_ref, out_ref):
    zs = zs_ref[...]
    zd = zd_ref[...]
    hm = (jnp.dot(e_ref[...], wem_ref[...],
                  preferred_element_type=jnp.float32)
          + bm1_ref[...])
    hm = hm + jnp.dot(zs, ws_ref[...], preferred_element_type=jnp.float32)
    hm = hm + jnp.dot(zd, wd_ref[...], preferred_element_type=jnp.float32)
    hm = hm + jnp.dot(jnp.abs(zs - zd), wa_ref[...],
                      preferred_element_type=jnp.float32)
    hm = hm + jnp.dot(zs * zd, wp_ref[...],
                      preferred_element_type=jnp.float32)
    hm = jnp.maximum(hm, 0.0)
    out_ref[...] = (
        jnp.dot(hm, wm2_ref[...], preferred_element_type=jnp.float32)
        + bm2_ref[...])


def _head(zs, zd, e, wem_t, bm1, ws_t, wd_t, wa_t, wp_t, wm2_t, bm2):
    return pl.pallas_call(
        _head_body,
        grid=(E // BEDGE,),
        in_specs=[
            pl.BlockSpec((BEDGE, 128), lambda i: (i, 0)),
            pl.BlockSpec((BEDGE, 128), lambda i: (i, 0)),
            pl.BlockSpec((BEDGE, 16), lambda i: (i, 0)),
            pl.BlockSpec((16, 128), lambda i: (0, 0)),
            pl.BlockSpec((1, 128), lambda i: (0, 0)),
            pl.BlockSpec((128, 128), lambda i: (0, 0)),
            pl.BlockSpec((128, 128), lambda i: (0, 0)),
            pl.BlockSpec((128, 128), lambda i: (0, 0)),
            pl.BlockSpec((128, 128), lambda i: (0, 0)),
            pl.BlockSpec((128, 2), lambda i: (0, 0)),
            pl.BlockSpec((1, 2), lambda i: (0, 0)),
        ],
        out_specs=pl.BlockSpec((BEDGE, 2), lambda i: (i, 0)),
        out_shape=jax.ShapeDtypeStruct((E, 2), jnp.float32),
    )(zs, zd, e, wem_t, bm1, ws_t, wd_t, wa_t, wp_t, wm2_t, bm2)


def kernel(x, edge_index, edge_attr, h_prev, params):
    p = params
    src = edge_index[0]
    dst = edge_index[1]
    e = edge_attr

    x_pad = jnp.zeros((NP, D), jnp.float32).at[:N].set(x)
    hp_pad = jnp.zeros((NP, H), jnp.float32).at[:N].set(h_prev)

    wg1_0, wg1_1 = p['Wg1_0'], p['Wg1_1']
    zpad = jnp.zeros((16, 64), jnp.float32)
    zpadb = jnp.zeros((64,), jnp.float32)
    w_ge0 = jnp.concatenate([wg1_0[:, 2 * D:].T, zpad], axis=1)
    b_ge0 = jnp.concatenate([p['bg1_0'], zpadb])
    w_ge1 = jnp.concatenate([wg1_1[:, 2 * D:].T, zpad], axis=1)
    b_ge1 = jnp.concatenate([p['bg1_1'], zpadb])
    wm1 = p['Wm1']
    ew0, ge0, ew1, ge1 = _edge_pre(e, p['W_edge0'].T, w_ge0,
                                   b_ge0.reshape(1, -1),
                                   p['W_edge1'].T, w_ge1,
                                   b_ge1.reshape(1, -1))

    cnt = _count_sc(dst)

    wgt0 = jnp.concatenate([wg1_0[:, D:2 * D].T, wg1_0[:, :D].T], axis=1)
    ts0, td0, gt0, slf0 = _node_pre(x_pad, p['W_src0'].T, p['W_dst0'].T,
                                    wgt0, p['W_self0'].T,
                                    p['b_self0'].reshape(1, -1))

    wg2_0 = p['Wg2_0'].reshape(64)
    bg2_0 = jnp.broadcast_to(p['bg2_0'].reshape(1), (16,))
    acc0 = _conv_sc(ts0, td0, gt0, ew0, ge0, src, dst, wg2_0, bg2_0)

    wgt1 = jnp.concatenate([wg1_1[:, D:2 * D].T, wg1_1[:, :D].T], axis=1)
    ts1, td1, gt1, slf1 = _combine0(acc0[0], acc0[1], cnt[0], cnt[1], slf0,
                                    p['ln_g0'].reshape(1, -1),
                                    p['ln_b0'].reshape(1, -1),
                                    p['W_src1'].T, p['W_dst1'].T, wgt1,
                                    p['W_self1'].T,
                                    p['b_self1'].reshape(1, -1))

    wg2_1 = p['Wg2_1'].reshape(64)
    bg2_1 = jnp.broadcast_to(p['bg2_1'].reshape(1), (16,))
    acc1 = _conv_sc(ts1, td1, gt1, ew1, ge1, src, dst, wg2_1, bg2_1)

    ht_pad = _combine1(acc1[0], acc1[1], cnt[0], cnt[1], slf1,
                       p['ln_g1'].reshape(1, -1),
                       p['ln_b1'].reshape(1, -1),
                       hp_pad,
                       p['W_ih'].T, p['b_ih'].reshape(1, -1),
                       p['W_hh'].T, p['b_hh'].reshape(1, -1))

    zs, zd = _clf_sc(ht_pad, src, dst)

    logits = _head(zs, zd, e,
                   wm1[:, 2 * H:2 * H + DE].T, p['bm1'].reshape(1, -1),
                   wm1[:, :H].T, wm1[:, H:2 * H].T,
                   wm1[:, 2 * H + DE:3 * H + DE].T, wm1[:, 3 * H + DE:].T,
                   p['Wm2'].T, p['bm2'].reshape(1, 2))
    return logits, ht_pad[:N]


# speedup vs baseline: 1.1468x; 1.0022x over previous
import functools

import jax
import jax.numpy as jnp
from jax import lax
from jax.experimental import pallas as pl
from jax.experimental.pallas import tpu as pltpu
from jax.experimental.pallas import tpu_sc as plsc

N = 10000
E = 320000
D = 128
DE = 16
H = 128

NP = 10240
NCORE = 2
NSUB = 16
NW = NCORE * NSUB
EPW = E // NW
CCH = 16
NCCH = EPW // CCH
CH = 80
NCHUNK = EPW // CH
NSLOT = 5
RPT = NP // NSUB

BN = 512
BEDGE = 2000

_MESH = plsc.VectorSubcoreMesh(core_axis_name="c", subcore_axis_name="s")
_SC_PARAMS = pltpu.CompilerParams(use_tc_tiling_on_sc=False)


def _conv_sc_body(goff,
                  ts_hbm, td_hbm, gt_hbm, ew_hbm, ge_hbm, src_hbm, dst_hbm,
                  wg2_hbm, bg2_hbm,
                  out_hbm,
                  idx_s0, idx_d0, idx_s1, idx_d1, sidx0, sidx1,
                  gs0, gd0, gj0, gi0, gs1, gd1, gj1, gi1,
                  ew0, ew1, ge0, ge1, msg0, msg1, wg2v, bg2v, acc,
                  semg0, semg1, semc0, semc1):
    c = lax.axis_index("c")
    s = lax.axis_index("s")
    wid = c * NSUB + s

    idx_s = [idx_s0, idx_s1]
    idx_d = [idx_d0, idx_d1]
    sidx = [sidx0, sidx1]
    gs = [gs0, gs1]
    gd = [gd0, gd1]
    gj = [gj0, gj1]
    gi = [gi0, gi1]
    ew = [ew0, ew1]
    ge = [ge0, ge1]
    msg = [msg0, msg1]
    semg = [semg0, semg1]
    semc = [semc0, semc1]

    zero16 = jnp.zeros((16,), jnp.float32)

    def zrow(i, carry):
        for k in range(8):
            msg0[i, pl.ds(16 * k, 16)] = zero16
        return carry

    lax.fori_loop(0, CCH, zrow, 0)
    rbase = s * RPT
    for t in range(RPT // CCH):
        pltpu.sync_copy(msg0, acc.at[pl.ds(rbase + t * CCH, CCH)])
    plsc.subcore_barrier()

    pltpu.sync_copy(wg2_hbm, wg2v)
    pltpu.sync_copy(bg2_hbm, bg2v)
    wg2k = [wg2v[pl.ds(16 * k, 16)] for k in range(4)]
    bg2 = bg2v[...]
    lane = lax.iota(jnp.int32, 16)
    gdn = lax.GatherDimensionNumbers(
        offset_dims=(), collapsed_slice_dims=(0,), start_index_map=(0,))
    perms = [jnp.bitwise_and(lane + sh, 15)[:, None] for sh in (8, 4, 2, 1)]

    ebase = wid * EPW

    def issue(ci, b):
        cb = ebase + ci * CCH
        pltpu.sync_copy(src_hbm.at[pl.ds(cb, CCH)], idx_s[b])
        pltpu.sync_copy(dst_hbm.at[pl.ds(cb, CCH)], idx_d[b])
        pltpu.async_copy(ts_hbm.at[idx_s[b]], gs[b], semg[b])
        pltpu.async_copy(td_hbm.at[idx_d[b]], gd[b], semg[b])
        pltpu.async_copy(gt_hbm.at[idx_s[b]], gj[b], semg[b])
        pltpu.async_copy(gt_hbm.at[idx_d[b]], gi[b], semg[b])
        pltpu.async_copy(ew_hbm.at[pl.ds(cb, CCH)], ew[b], semg[b])
        pltpu.async_copy(ge_hbm.at[pl.ds(cb, CCH)], ge[b], semg[b])

    def wait_gathers(b):
        pltpu.make_async_copy(ts_hbm.at[idx_s[b]], gs[b], semg[b]).wait()
        pltpu.make_async_copy(td_hbm.at[idx_d[b]], gd[b], semg[b]).wait()
        pltpu.make_async_copy(gt_hbm.at[idx_s[b]], gj[b], semg[b]).wait()
        pltpu.make_async_copy(gt_hbm.at[idx_d[b]], gi[b], semg[b]).wait()
        pltpu.make_async_copy(ew_hbm.at[pl.ds(0, CCH)], ew[b], semg[b]).wait()
        pltpu.make_async_copy(ge_hbm.at[pl.ds(0, CCH)], ge[b], semg[b]).wait()

    def wait_scatter(b):
        pltpu.make_async_copy(msg[b], acc.at[sidx[b]], semc[b]).wait()

    def copy_idx(b):
        sidx[b][pl.ds(0, 16)] = idx_d[b][pl.ds(0, 16)]

    def edge_one(e2, b):
        gh_acc = None
        for k in range(4):
            ghk = (gj[b][e2, pl.ds(16 * k, 16)]
                   + gi[b][e2, pl.ds(64 + 16 * k, 16)]
                   + ge[b][e2, pl.ds(goff + 16 * k, 16)])
            ghk = jnp.maximum(ghk, 0.0) * wg2k[k]
            gh_acc = ghk if gh_acc is None else gh_acc + ghk
        v = gh_acc
        for pm in perms:
            v = v + lax.gather(
                v, pm, gdn, slice_sizes=(1,),
                mode=lax.GatherScatterMode.PROMISE_IN_BOUNDS)
        tv = v + bg2
        gate = 1.0 / (1.0 + jnp.exp(-tv))
        for k in range(8):
            raw = (gs[b][e2, pl.ds(16 * k, 16)]
                   + gd[b][e2, pl.ds(16 * k, 16)]
                   + ew[b][e2, pl.ds(16 * k, 16)])
            msg[b][e2, pl.ds(16 * k, 16)] = jnp.maximum(raw, 0.0) * gate

    def halfstep(i, ci, b, last):
        wait_gathers(b)

        @pl.when(i > 0)
        def _():
            wait_scatter(b)

        @plsc.parallel_loop(0, CCH, 1, unroll=4)
        def _(e2):
            edge_one(e2, b)

        copy_idx(b)
        pltpu.async_copy(msg[b], acc.at[sidx[b]], semc[b], add=True)

        @pl.when(i < last)
        def _():
            issue(ci + 2, b)

    issue(0, 0)
    issue(1, 1)

    npair = NCCH // 2

    def chunk_pair(i, carry):
        halfstep(i, 2 * i, 0, npair)
        halfstep(i, 2 * i + 1, 1, npair - 1)
        return carry

    lax.fori_loop(0, npair, chunk_pair, 0)
    halfstep(npair, NCCH - 1, 0, -1)
    wait_scatter(0)
    wait_scatter(1)
    plsc.subcore_barrier()
    pltpu.sync_copy(acc.at[pl.ds(rbase, RPT)], out_hbm.at[c, pl.ds(rbase, RPT)])


def _make_conv_sc(goff):
    return pl.kernel(
        functools.partial(_conv_sc_body, goff),
        out_type=jax.ShapeDtypeStruct((NCORE, NP, 128), jnp.float32),
        mesh=_MESH,
        scratch_types=(
            [pltpu.VMEM((CCH,), jnp.int32) for _ in range(6)]
            + [pltpu.VMEM((CCH, 128), jnp.float32) for _ in range(14)]
            + [pltpu.VMEM((64,), jnp.float32), pltpu.VMEM((16,), jnp.float32),
               pltpu.VMEM_SHARED((NP, 128), jnp.float32)]
            + [pltpu.SemaphoreType.DMA for _ in range(4)]
        ),
        compiler_params=_SC_PARAMS,
    )


_conv_sc0 = _make_conv_sc(0)
_conv_sc1 = _make_conv_sc(64)


def _count_sc_body(dst_hbm, out_hbm, idx_d, ones, acc, sem0):
    c = lax.axis_index("c")
    s = lax.axis_index("s")
    wid = c * NSUB + s

    zero16 = jnp.zeros((16,), jnp.float32)

    def zrow(i, carry):
        ones[i, pl.ds(0, 16)] = zero16
        return carry

    lax.fori_loop(0, CH, zrow, 0)
    rbase = s * RPT
    for t in range(RPT // CH):
        pltpu.sync_copy(ones, acc.at[pl.ds(rbase + t * CH, CH)])
    plsc.subcore_barrier()

    lane = lax.iota(jnp.int32, 16)
    cnt_vec = jnp.where(lane == 0, 1.0, 0.0).astype(jnp.float32)

    def orow(i, carry):
        ones[i, pl.ds(0, 16)] = cnt_vec
        return carry

    lax.fori_loop(0, CH, orow, 0)

    ebase = wid * EPW

    def chunk_body(i, carry):
        cb = ebase + i * CH
        pltpu.sync_copy(dst_hbm.at[pl.ds(cb, CH)], idx_d)
        pltpu.sync_copy(ones, acc.at[idx_d], add=True)
        return carry

    lax.fori_loop(0, NCHUNK, chunk_body, 0)
    plsc.subcore_barrier()
    pltpu.sync_copy(acc.at[pl.ds(rbase, RPT)], out_hbm.at[c, pl.ds(rbase, RPT)])


_count_sc = pl.kernel(
    _count_sc_body,
    out_type=jax.ShapeDtypeStruct((NCORE, NP, 16), jnp.float32),
    mesh=_MESH,
    scratch_types=[
        pltpu.VMEM((CH,), jnp.int32),
        pltpu.VMEM((CH, 16), jnp.float32),
        pltpu.VMEM_SHARED((NP, 16), jnp.float32),
        pltpu.SemaphoreType.DMA,
    ],
    compiler_params=_SC_PARAMS,
)


def _clf_sc_body(ht_hbm, src_hbm, dst_hbm, zs_hbm, zd_hbm,
                 idx_s0, idx_d0, idx_s1, idx_d1, idx_s2, idx_d2,
                 idx_s3, idx_d3, idx_s4, idx_d4,
                 u0, v0, u1, v1, u2, v2, u3, v3, u4, v4,
                 semg0, semg1, semg2, semg3, semg4,
                 semw0, semw1, semw2, semw3, semw4):
    c = lax.axis_index("c")
    s = lax.axis_index("s")
    wid = c * NSUB + s
    ebase = wid * EPW

    idx_s = [idx_s0, idx_s1, idx_s2, idx_s3, idx_s4]
    idx_d = [idx_d0, idx_d1, idx_d2, idx_d3, idx_d4]
    u = [u0, u1, u2, u3, u4]
    v = [v0, v1, v2, v3, v4]
    semg = [semg0, semg1, semg2, semg3, semg4]
    semw = [semw0, semw1, semw2, semw3, semw4]

    def issue(ci, b):
        cb = ebase + ci * CH
        pltpu.sync_copy(src_hbm.at[pl.ds(cb, CH)], idx_s[b])
        pltpu.sync_copy(dst_hbm.at[pl.ds(cb, CH)], idx_d[b])
        pltpu.async_copy(ht_hbm.at[idx_s[b]], u[b], semg[b])
        pltpu.async_copy(ht_hbm.at[idx_d[b]], v[b], semg[b])

    def wait_gathers(b):
        pltpu.make_async_copy(ht_hbm.at[idx_s[b]], u[b], semg[b]).wait()
        pltpu.make_async_copy(ht_hbm.at[idx_d[b]], v[b], semg[b]).wait()

    def start_writes(ci, b):
        cb = ebase + ci * CH
        pltpu.async_copy(u[b], zs_hbm.at[pl.ds(cb, CH)], semw[b])
        pltpu.async_copy(v[b], zd_hbm.at[pl.ds(cb, CH)], semw[b])

    def wait_writes(ci, b):
        cb = ebase + ci * CH
        pltpu.make_async_copy(u[b], zs_hbm.at[pl.ds(cb, CH)], semw[b]).wait()
        pltpu.make_async_copy(v[b], zd_hbm.at[pl.ds(cb, CH)], semw[b]).wait()

    for b in range(NSLOT):
        issue(b, b)

    nq = NCHUNK // NSLOT

    def quint(i, carry):
        for b in range(NSLOT):
            wait_gathers(b)
            start_writes(i * NSLOT + b, b)
        for b in range(NSLOT):
            @pl.when(i < nq - 1)
            def _(b=b):
                wait_writes(i * NSLOT + b, b)
                issue((i + 1) * NSLOT + b, b)
        return carry

    lax.fori_loop(0, nq, quint, 0)
    for b in range(NSLOT):
        wait_writes((nq - 1) * NSLOT + b, b)


_clf_sc = pl.kernel(
    _clf_sc_body,
    out_type=(jax.ShapeDtypeStruct((E, 128), jnp.float32),
              jax.ShapeDtypeStruct((E, 128), jnp.float32)),
    mesh=_MESH,
    scratch_types=(
        [pltpu.VMEM((CH,), jnp.int32) for _ in range(2 * NSLOT)]
        + [pltpu.VMEM((CH, 128), jnp.float32) for _ in range(2 * NSLOT)]
        + [pltpu.SemaphoreType.DMA for _ in range(2 * NSLOT)]
    ),
    compiler_params=_SC_PARAMS,
)


def _node_pre_body(x_ref, w1_ref, w2_ref, wg_ref, w3_ref, b3_ref,
                   ts_ref, td_ref, gt_ref, self_ref):
    x = x_ref[...]
    ts_ref[...] = jnp.dot(x, w1_ref[...], preferred_element_type=jnp.float32)
    td_ref[...] = jnp.dot(x, w2_ref[...], preferred_element_type=jnp.float32)
    gt_ref[...] = jnp.dot(x, wg_ref[...], preferred_element_type=jnp.float32)
    self_ref[...] = (
        jnp.dot(x, w3_ref[...], preferred_element_type=jnp.float32)
        + b3_ref[...])


def _node_pre(x, w1, w2, wg, w3, b3):
    return pl.pallas_call(
        _node_pre_body,
        grid=(NP // BN,),
        in_specs=[
            pl.BlockSpec((BN, 128), lambda i: (i, 0)),
            pl.BlockSpec((128, 128), lambda i: (0, 0)),
            pl.BlockSpec((128, 128), lambda i: (0, 0)),
            pl.BlockSpec((128, 128), lambda i: (0, 0)),
            pl.BlockSpec((128, 128), lambda i: (0, 0)),
            pl.BlockSpec((1, 128), lambda i: (0, 0)),
        ],
        out_specs=[
            pl.BlockSpec((BN, 128), lambda i: (i, 0)),
            pl.BlockSpec((BN, 128), lambda i: (i, 0)),
            pl.BlockSpec((BN, 128), lambda i: (i, 0)),
            pl.BlockSpec((BN, 128), lambda i: (i, 0)),
        ],
        out_shape=[
            jax.ShapeDtypeStruct((NP, 128), jnp.float32),
            jax.ShapeDtypeStruct((NP, 128), jnp.float32),
            jax.ShapeDtypeStruct((NP, 128), jnp.float32),
            jax.ShapeDtypeStruct((NP, 128), jnp.float32),
        ],
    )(x, w1, w2, wg, w3, b3)


def _edge_pre_body(e_ref, we0_ref, we1_ref, wg_ref, bg_ref,
                   ew0_ref, ew1_ref, ge_ref):
    e = e_ref[...]
    ew0_ref[...] = jnp.dot(e, we0_ref[...],
                           preferred_element_type=jnp.float32)
    ew1_ref[...] = jnp.dot(e, we1_ref[...],
                           preferred_element_type=jnp.float32)
    ge_ref[...] = (jnp.dot(e, wg_ref[...],
                           preferred_element_type=jnp.float32)
                   + bg_ref[...])


def _edge_pre(e, we0, we1, wg, bg):
    return pl.pallas_call(
        _edge_pre_body,
        grid=(E // BEDGE,),
        in_specs=[
            pl.BlockSpec((BEDGE, 16), lambda i: (i, 0)),
            pl.BlockSpec((16, 128), lambda i: (0, 0)),
            pl.BlockSpec((16, 128), lambda i: (0, 0)),
            pl.BlockSpec((16, 128), lambda i: (0, 0)),
            pl.BlockSpec((1, 128), lambda i: (0, 0)),
        ],
        out_specs=[
            pl.BlockSpec((BEDGE, 128), lambda i: (i, 0)),
            pl.BlockSpec((BEDGE, 128), lambda i: (i, 0)),
            pl.BlockSpec((BEDGE, 128), lambda i: (i, 0)),
        ],
        out_shape=[
            jax.ShapeDtypeStruct((E, 128), jnp.float32),
            jax.ShapeDtypeStruct((E, 128), jnp.float32),
            jax.ShapeDtypeStruct((E, 128), jnp.float32),
        ],
    )(e, we0, we1, wg, bg)


def _finish_node(a0, a1, c0, c1, slf, g, b):
    sums = a0 + a1
    cnt = jnp.sum(c0 + c1, axis=1, keepdims=True)
    agg = sums / jnp.maximum(cnt, 1.0)
    o = agg + slf
    m = jnp.mean(o, axis=-1, keepdims=True)
    v = jnp.mean((o - m) ** 2, axis=-1, keepdims=True)
    hn = (o - m) / jnp.sqrt(v + 1e-5) * g + b
    return jnp.maximum(hn, 0.0)


def _combine0_body(a0_ref, a1_ref, c0_ref, c1_ref, self_ref, g_ref, b_ref,
                   w1_ref, w2_ref, wg_ref, w3_ref, b3_ref,
                   ts_ref, td_ref, gt_ref, self1_ref):
    h = _finish_node(a0_ref[...], a1_ref[...], c0_ref[...], c1_ref[...],
                     self_ref[...], g_ref[...], b_ref[...])
    ts_ref[...] = jnp.dot(h, w1_ref[...], preferred_element_type=jnp.float32)
    td_ref[...] = jnp.dot(h, w2_ref[...], preferred_element_type=jnp.float32)
    gt_ref[...] = jnp.dot(h, wg_ref[...], preferred_element_type=jnp.float32)
    self1_ref[...] = (
        jnp.dot(h, w3_ref[...], preferred_element_type=jnp.float32)
        + b3_ref[...])


def _combine0(a0, a1, c0, c1, slf, g, b, w1, w2, wg, w3, b3):
    return pl.pallas_call(
        _combine0_body,
        grid=(NP // BN,),
        in_specs=[
            pl.BlockSpec((BN, 128), lambda i: (i, 0)),
            pl.BlockSpec((BN, 128), lambda i: (i, 0)),
            pl.BlockSpec((BN, 16), lambda i: (i, 0)),
            pl.BlockSpec((BN, 16), lambda i: (i, 0)),
            pl.BlockSpec((BN, 128), lambda i: (i, 0)),
            pl.BlockSpec((1, 128), lambda i: (0, 0)),
            pl.BlockSpec((1, 128), lambda i: (0, 0)),
            pl.BlockSpec((128, 128), lambda i: (0, 0)),
            pl.BlockSpec((128, 128), lambda i: (0, 0)),
            pl.BlockSpec((128, 128), lambda i: (0, 0)),
            pl.BlockSpec((128, 128), lambda i: (0, 0)),
            pl.BlockSpec((1, 128), lambda i: (0, 0)),
        ],
        out_specs=[
            pl.BlockSpec((BN, 128), lambda i: (i, 0)),
            pl.BlockSpec((BN, 128), lambda i: (i, 0)),
            pl.BlockSpec((BN, 128), lambda i: (i, 0)),
            pl.BlockSpec((BN, 128), lambda i: (i, 0)),
        ],
        out_shape=[
            jax.ShapeDtypeStruct((NP, 128), jnp.float32),
            jax.ShapeDtypeStruct((NP, 128), jnp.float32),
            jax.ShapeDtypeStruct((NP, 128), jnp.float32),
            jax.ShapeDtypeStruct((NP, 128), jnp.float32),
        ],
    )(a0, a1, c0, c1, slf, g, b, w1, w2, wg, w3, b3)


def _combine1_body(a0_ref, a1_ref, c0_ref, c1_ref, self_ref, g_ref, b_ref,
                   hp_ref, wih_ref, bih_ref, whh_ref, bhh_ref,
                   ht_ref):
    h = _finish_node(a0_ref[...], a1_ref[...], c0_ref[...], c1_ref[...],
                     self_ref[...], g_ref[...], b_ref[...])
    hp = hp_ref[...]
    gi = jnp.dot(h, wih_ref[...], preferred_element_type=jnp.float32) \
        + bih_ref[...]
    gh = jnp.dot(hp, whh_ref[...], preferred_element_type=jnp.float32) \
        + bhh_ref[...]
    r = jax.nn.sigmoid(gi[:, :128] + gh[:, :128])
    z = jax.nn.sigmoid(gi[:, 128:256] + gh[:, 128:256])
    n = jnp.tanh(gi[:, 256:384] + r * gh[:, 256:384])
    ht_ref[...] = (1.0 - z) * n + z * hp


def _combine1(a0, a1, c0, c1, slf, g, b, hp, wih, bih, whh, bhh):
    return pl.pallas_call(
        _combine1_body,
        grid=(NP // BN,),
        in_specs=[
            pl.BlockSpec((BN, 128), lambda i: (i, 0)),
            pl.BlockSpec((BN, 128), lambda i: (i, 0)),
            pl.BlockSpec((BN, 16), lambda i: (i, 0)),
            pl.BlockSpec((BN, 16), lambda i: (i, 0)),
            pl.BlockSpec((BN, 128), lambda i: (i, 0)),
            pl.BlockSpec((1, 128), lambda i: (0, 0)),
            pl.BlockSpec((1, 128), lambda i: (0, 0)),
            pl.BlockSpec((BN, 128), lambda i: (i, 0)),
            pl.BlockSpec((128, 384), lambda i: (0, 0)),
            pl.BlockSpec((1, 384), lambda i: (0, 0)),
            pl.BlockSpec((128, 384), lambda i: (0, 0)),
            pl.BlockSpec((1, 384), lambda i: (0, 0)),
        ],
        out_specs=pl.BlockSpec((BN, 128), lambda i: (i, 0)),
        out_shape=jax.ShapeDtypeStruct((NP, 128), jnp.float32),
    )(a0, a1, c0, c1, slf, g, b, hp, wih, bih, whh, bhh)


def _head_body(zs_ref, zd_ref, e_ref, wem_ref, bm1_ref,
               ws_ref, wd_ref, wa_ref, wp_ref,
               wm2_ref, bm2_ref, out_ref):
    zs = zs_ref[...]
    zd = zd_ref[...]
    bf = jnp.bfloat16
    hm = (jnp.dot(e_ref[...], wem_ref[...],
                  preferred_element_type=jnp.float32)
          + bm1_ref[...])
    hm = hm + jnp.dot(zs.astype(bf), ws_ref[...].astype(bf),
                      preferred_element_type=jnp.float32)
    hm = hm + jnp.dot(zd.astype(bf), wd_ref[...].astype(bf),
                      preferred_element_type=jnp.float32)
    hm = hm + jnp.dot(jnp.abs(zs - zd).astype(bf), wa_ref[...].astype(bf),
                      preferred_element_type=jnp.float32)
    hm = hm + jnp.dot((zs * zd).astype(bf), wp_ref[...].astype(bf),
                      preferred_element_type=jnp.float32)
    hm = jnp.maximum(hm, 0.0)
    out_ref[...] = (
        jnp.dot(hm, wm2_ref[...], preferred_element_type=jnp.float32)
        + bm2_ref[...])


def _head(zs, zd, e, wem_t, bm1, ws_t, wd_t, wa_t, wp_t, wm2_t, bm2):
    return pl.pallas_call(
        _head_body,
        grid=(E // BEDGE,),
        in_specs=[
            pl.BlockSpec((BEDGE, 128), lambda i: (i, 0)),
            pl.BlockSpec((BEDGE, 128), lambda i: (i, 0)),
            pl.BlockSpec((BEDGE, 16), lambda i: (i, 0)),
            pl.BlockSpec((16, 128), lambda i: (0, 0)),
            pl.BlockSpec((1, 128), lambda i: (0, 0)),
            pl.BlockSpec((128, 128), lambda i: (0, 0)),
            pl.BlockSpec((128, 128), lambda i: (0, 0)),
            pl.BlockSpec((128, 128), lambda i: (0, 0)),
            pl.BlockSpec((128, 128), lambda i: (0, 0)),
            pl.BlockSpec((128, 2), lambda i: (0, 0)),
            pl.BlockSpec((1, 2), lambda i: (0, 0)),
        ],
        out_specs=pl.BlockSpec((BEDGE, 2), lambda i: (i, 0)),
        out_shape=jax.ShapeDtypeStruct((E, 2), jnp.float32),
    )(zs, zd, e, wem_t, bm1, ws_t, wd_t, wa_t, wp_t, wm2_t, bm2)


def kernel(x, edge_index, edge_attr, h_prev, params):
    p = params
    src = edge_index[0]
    dst = edge_index[1]
    e = edge_attr

    x_pad = jnp.zeros((NP, D), jnp.float32).at[:N].set(x)
    hp_pad = jnp.zeros((NP, H), jnp.float32).at[:N].set(h_prev)

    wg1_0, wg1_1 = p['Wg1_0'], p['Wg1_1']
    w_ge = jnp.concatenate([wg1_0[:, 2 * D:].T, wg1_1[:, 2 * D:].T], axis=1)
    b_ge = jnp.concatenate([p['bg1_0'], p['bg1_1']])
    wm1 = p['Wm1']
    ew0, ew1, ge01 = _edge_pre(e, p['W_edge0'].T, p['W_edge1'].T,
                               w_ge, b_ge.reshape(1, -1))

    cnt = _count_sc(dst)

    wgt0 = jnp.concatenate([wg1_0[:, D:2 * D].T, wg1_0[:, :D].T], axis=1)
    ts0, td0, gt0, slf0 = _node_pre(x_pad, p['W_src0'].T, p['W_dst0'].T,
                                    wgt0, p['W_self0'].T,
                                    p['b_self0'].reshape(1, -1))

    wg2_0 = p['Wg2_0'].reshape(64)
    bg2_0 = jnp.broadcast_to(p['bg2_0'].reshape(1), (16,))
    acc0 = _conv_sc0(ts0, td0, gt0, ew0, ge01, src, dst, wg2_0, bg2_0)

    wgt1 = jnp.concatenate([wg1_1[:, D:2 * D].T, wg1_1[:, :D].T], axis=1)
    ts1, td1, gt1, slf1 = _combine0(acc0[0], acc0[1], cnt[0], cnt[1], slf0,
                                    p['ln_g0'].reshape(1, -1),
                                    p['ln_b0'].reshape(1, -1),
                                    p['W_src1'].T, p['W_dst1'].T, wgt1,
                                    p['W_self1'].T,
                                    p['b_self1'].reshape(1, -1))

    wg2_1 = p['Wg2_1'].reshape(64)
    bg2_1 = jnp.broadcast_to(p['bg2_1'].reshape(1), (16,))
    acc1 = _conv_sc1(ts1, td1, gt1, ew1, ge01, src, dst, wg2_1, bg2_1)

    ht_pad = _combine1(acc1[0], acc1[1], cnt[0], cnt[1], slf1,
                       p['ln_g1'].reshape(1, -1),
                       p['ln_b1'].reshape(1, -1),
                       hp_pad,
                       p['W_ih'].T, p['b_ih'].reshape(1, -1),
                       p['W_hh'].T, p['b_hh'].reshape(1, -1))

    zs, zd = _clf_sc(ht_pad, src, dst)

    logits = _head(zs, zd, e,
                   wm1[:, 2 * H:2 * H + DE].T, p['bm1'].reshape(1, -1),
                   wm1[:, :H].T, wm1[:, H:2 * H].T,
                   wm1[:, 2 * H + DE:3 * H + DE].T, wm1[:, 3 * H + DE:].T,
                   p['Wm2'].T, p['bm2'].reshape(1, 2))
    return logits, ht_pad[:N]
